# Initial kernel scaffold; baseline (speedup 1.0000x reference)
import jax
import jax.numpy as jnp
from jax.experimental import pallas as pl


def kernel(x, t, linker_batch, linker_graph_ptr, linker_node_type, linker_edge_index, left_x, left_edge_index, left_batch, right_x, right_edge_index, right_batch, params):
    raise NotImplementedError("write your pallas kernel here")


# speedup vs baseline: 4.5500x; 4.5500x over previous
import functools
import math

import jax
import jax.numpy as jnp
from jax import lax
from jax.experimental import pallas as pl
from jax.experimental.pallas import tpu as pltpu
from jax.experimental.pallas import tpu_sc as plsc

N = 50000
E = 800000
G = 128
H = 64
HH = 32
NC, NS, LANES = 2, 16, 16

EPT = E // NS
EPT_PAD = 51200
ROWS_PT = EPT_PAD // 128
GR = 4
NGRP = ROWS_PT // GR
NP = 50048
TROWS = NP // NS
CHUNKS = [(k * 128, 128) for k in range(24)] + [(24 * 128, 56)]
SRC_PAD = N
DST_PAD = N

BLK = 2000
GRID = N // BLK

_mesh = plsc.VectorSubcoreMesh(core_axis_name="c", subcore_axis_name="s",
                               num_cores=NC, num_subcores=NS)
_CP = pltpu.CompilerParams(use_tc_tiling_on_sc=False)


def _zero_vmem_rows(buf, nrows, width):
    def body(i, _):
        for w0 in range(0, width, LANES):
            buf[i, pl.ds(w0, LANES)] = jnp.zeros((LANES,), jnp.float32)
        return 0
    lax.fori_loop(0, nrows, body, 0)


def _agg_body(h0, h1, src_r, dst_r, o0, o1,
              src_v, dst_v, rows_v, zbuf, acc, gsem, ssem):
    c = lax.axis_index("c")
    s = lax.axis_index("s")

    def run(h_ref, out_ref):
        _zero_vmem_rows(zbuf, 128, HH)
        base = s * TROWS
        for off, ln in CHUNKS:
            pltpu.sync_copy(zbuf.at[pl.ds(0, ln)],
                            acc.at[pl.ds(base + off, ln)])
        plsc.subcore_barrier()

        row0 = s * ROWS_PT

        def grp(g, _):
            r = row0 + g * GR
            pltpu.sync_copy(src_r.at[pl.ds(r, GR)], src_v)
            pltpu.sync_copy(dst_r.at[pl.ds(r, GR)], dst_v)
            ds_ = []
            for j in range(GR):
                ds_.append(pltpu.async_copy(h_ref.at[src_v.at[j]],
                                            rows_v.at[j], gsem))
            for d in ds_:
                d.wait()
            ds_ = []
            for j in range(GR):
                ds_.append(pltpu.async_copy(rows_v.at[j], acc.at[dst_v.at[j]],
                                            ssem, add=True))
            for d in ds_:
                d.wait()
            return 0

        lax.fori_loop(0, NGRP, grp, 0)
        plsc.subcore_barrier()

        for off, ln in CHUNKS:
            pltpu.sync_copy(acc.at[pl.ds(base + off, ln)],
                            zbuf.at[pl.ds(0, ln)])
            pltpu.sync_copy(zbuf.at[pl.ds(0, ln)],
                            out_ref.at[pl.ds(base + off, ln)])

    @pl.when(c == 0)
    def _():
        run(h0, o0)

    @pl.when(c == 1)
    def _():
        run(h1, o1)


_agg_call = pl.kernel(
    _agg_body,
    out_type=(jax.ShapeDtypeStruct((NP, HH), jnp.float32),
              jax.ShapeDtypeStruct((NP, HH), jnp.float32)),
    mesh=_mesh,
    compiler_params=_CP,
    scratch_types=[
        pltpu.VMEM((GR, 128), jnp.int32),
        pltpu.VMEM((GR, 128), jnp.int32),
        pltpu.VMEM((GR, 128, HH), jnp.float32),
        pltpu.VMEM((128, HH), jnp.float32),
        pltpu.VMEM_SHARED((NP, HH), jnp.float32),
        pltpu.SemaphoreType.DMA,
        pltpu.SemaphoreType.DMA,
    ],
)


def _deg_body(dst_l, dst_f, out_l, out_f, dst_v, ones_v, zbuf, acc, sem):
    c = lax.axis_index("c")
    s = lax.axis_index("s")

    def run(dst_r, out_ref):
        io = lax.broadcasted_iota(jnp.int32, (LANES,), 0)
        one_row = jnp.where(io == 0, 1.0, 0.0).astype(jnp.float32)

        def seto(i, _):
            ones_v[i, pl.ds(0, LANES)] = one_row
            return 0
        lax.fori_loop(0, 128, seto, 0)

        _zero_vmem_rows(zbuf, 128, LANES)
        base = s * TROWS
        for off, ln in CHUNKS:
            pltpu.sync_copy(zbuf.at[pl.ds(0, ln)],
                            acc.at[pl.ds(base + off, ln)])
        plsc.subcore_barrier()

        row0 = s * ROWS_PT

        def grp(g, _):
            r = row0 + g * GR
            pltpu.sync_copy(dst_r.at[pl.ds(r, GR)], dst_v)
            ds_ = []
            for j in range(GR):
                ds_.append(pltpu.async_copy(ones_v, acc.at[dst_v.at[j]], sem,
                                            add=True))
            for d in ds_:
                d.wait()
            return 0

        lax.fori_loop(0, NGRP, grp, 0)
        plsc.subcore_barrier()

        for off, ln in CHUNKS:
            pltpu.sync_copy(acc.at[pl.ds(base + off, ln)],
                            zbuf.at[pl.ds(0, ln)])
            pltpu.sync_copy(zbuf.at[pl.ds(0, ln)],
                            out_ref.at[pl.ds(base + off, ln)])

    @pl.when(c == 0)
    def _():
        run(dst_l, out_l)

    @pl.when(c == 1)
    def _():
        run(dst_f, out_f)


_deg_call = pl.kernel(
    _deg_body,
    out_type=(jax.ShapeDtypeStruct((NP, LANES), jnp.float32),
              jax.ShapeDtypeStruct((NP, LANES), jnp.float32)),
    mesh=_mesh,
    compiler_params=_CP,
    scratch_types=[
        pltpu.VMEM((GR, 128), jnp.int32),
        pltpu.VMEM((128, LANES), jnp.float32),
        pltpu.VMEM((128, LANES), jnp.float32),
        pltpu.VMEM_SHARED((NP, LANES), jnp.float32),
        pltpu.SemaphoreType.DMA,
    ],
)


def _prep_edges(src, dst):
    s = src.astype(jnp.int32).reshape(NS, EPT)
    d = dst.astype(jnp.int32).reshape(NS, EPT)
    s = jnp.pad(s, ((0, 0), (0, EPT_PAD - EPT)), constant_values=SRC_PAD)
    d = jnp.pad(d, ((0, 0), (0, EPT_PAD - EPT)), constant_values=DST_PAD)
    return s.reshape(NS * ROWS_PT, 128), d.reshape(NS * ROWS_PT, 128)


def _rowspec(w):
    return pl.BlockSpec((BLK, w), lambda i: (i, 0))


def _fullspec(shape):
    return pl.BlockSpec(shape, lambda i: (0,) * len(shape))


def _ln_relu(y, g, b):
    m = jnp.mean(y, axis=-1, keepdims=True)
    v = jnp.mean((y - m) ** 2, axis=-1, keepdims=True)
    return jnp.maximum((y - m) / jnp.sqrt(v + 1e-5) * g + b, 0.0)


def _conv_tc_body(has_ctx, h0, h1, a0, a1, deg, batch, gctx, W, b, g, bb,
                  o0, o1):
    h = jnp.concatenate([h0[...], h1[...]], axis=1)
    a = jnp.concatenate([a0[...], a1[...]], axis=1)
    d = jnp.maximum(deg[...][:, 0:1], 1.0)
    y = jnp.dot(h + a / d, W[...], preferred_element_type=jnp.float32) + b[...]
    if has_ctx:
        io = lax.broadcasted_iota(jnp.int32, (BLK, G), 1)
        oh = (batch[...] == io).astype(jnp.float32)
        y = y + jnp.dot(oh, gctx[...], preferred_element_type=jnp.float32)
    r = _ln_relu(y, g[...], bb[...])
    o0[...] = r[:, :HH]
    o1[...] = r[:, HH:]


def _make_conv_tc(has_ctx):
    in_specs = [_rowspec(HH)] * 4 + [
        _rowspec(LANES), _rowspec(1), _fullspec((G, H)),
        _fullspec((H, H)), _fullspec((1, H)), _fullspec((1, H)),
        _fullspec((1, H))]
    return pl.pallas_call(
        functools.partial(_conv_tc_body, has_ctx),
        grid=(GRID,),
        in_specs=in_specs,
        out_specs=(_rowspec(HH), _rowspec(HH)),
        out_shape=(jax.ShapeDtypeStruct((NP, HH), jnp.float32),
                   jax.ShapeDtypeStruct((NP, HH), jnp.float32)),
    )


_conv_tc_ctx = _make_conv_tc(True)
_conv_tc_noctx = _make_conv_tc(False)


def _frag_in_body(fx, W, b, o0, o1):
    y = jnp.dot(fx[...], W[...], preferred_element_type=jnp.float32) + b[...]
    o0[...] = y[:, :HH]
    o1[...] = y[:, HH:]


_frag_in = pl.pallas_call(
    _frag_in_body,
    grid=(GRID,),
    in_specs=[_rowspec(4), _fullspec((4, H)), _fullspec((1, H))],
    out_specs=(_rowspec(HH), _rowspec(HH)),
    out_shape=(jax.ShapeDtypeStruct((NP, HH), jnp.float32),
               jax.ShapeDtypeStruct((NP, HH), jnp.float32)),
)


def _linker_in_body(x, nt, batch, gctx, W, b, o0, o1):
    xv = x[...]
    ntv = nt[...]
    y = jnp.dot(xv, W[...][:4, :], preferred_element_type=jnp.float32) + b[...]
    ntc = jnp.clip(ntv, 0, 2)
    for k in range(3):
        y = y + (ntc == k).astype(jnp.float32) * W[...][4 + k:5 + k, :]
    y = y + (ntv > 0).astype(jnp.float32) * W[...][7:8, :]
    io = lax.broadcasted_iota(jnp.int32, (BLK, G), 1)
    oh = (batch[...] == io).astype(jnp.float32)
    y = y + jnp.dot(oh, gctx[...], preferred_element_type=jnp.float32)
    o0[...] = y[:, :HH]
    o1[...] = y[:, HH:]


_linker_in = pl.pallas_call(
    _linker_in_body,
    grid=(GRID,),
    in_specs=[_rowspec(4), _rowspec(1), _rowspec(1), _fullspec((G, H)),
              _fullspec((8, H)), _fullspec((1, H))],
    out_specs=(_rowspec(HH), _rowspec(HH)),
    out_shape=(jax.ShapeDtypeStruct((NP, HH), jnp.float32),
               jax.ShapeDtypeStruct((NP, HH), jnp.float32)),
)


def _pool_body(h0, h1, batch, sums, cnt):
    i = pl.program_id(0)

    @pl.when(i == 0)
    def _():
        sums[...] = jnp.zeros_like(sums)
        cnt[...] = jnp.zeros_like(cnt)

    h = jnp.concatenate([h0[...], h1[...]], axis=1)
    io = lax.broadcasted_iota(jnp.int32, (BLK, 2 * G), 1)
    oh = (batch[...] == io).astype(jnp.float32)
    dn = (((0,), (0,)), ((), ()))
    sums[...] += lax.dot_general(oh, h, dn,
                                 preferred_element_type=jnp.float32)
    cnt[...] += lax.dot_general(oh, jnp.ones((BLK, 8), jnp.float32), dn,
                                preferred_element_type=jnp.float32)


_pool_tc = pl.pallas_call(
    _pool_body,
    grid=(GRID,),
    in_specs=[_rowspec(HH), _rowspec(HH), _rowspec(1)],
    out_specs=(_fullspec((2 * G, H)), _fullspec((2 * G, 8))),
    out_shape=(jax.ShapeDtypeStruct((2 * G, H), jnp.float32),
               jax.ShapeDtypeStruct((2 * G, 8), jnp.float32)),
)


def _gctx_body(sums, cnt, fW, fb, t, tW1, tb1, tW2, tb2, cW1, cb1, cW2, cb2,
               out):
    pooled = sums[...] / jnp.maximum(cnt[...][:, 0:1], 1.0)
    fctx = jnp.dot(pooled, fW[...], preferred_element_type=jnp.float32) + fb[...]
    left = fctx[:G, :]
    right = fctx[G:, :]
    j = lax.broadcasted_iota(jnp.int32, (1, H), 1).astype(jnp.float32)
    freqs = jnp.exp(-math.log(10000.0) * j / 64.0)
    a = t[...] * freqs
    te = jnp.concatenate([jnp.sin(a), jnp.cos(a)], axis=1)
    z = jnp.dot(te, tW1[...], preferred_element_type=jnp.float32) + tb1[...]
    z = z * lax.logistic(z)
    th = jnp.dot(z, tW2[...], preferred_element_type=jnp.float32) + tb2[...]
    ci = jnp.concatenate([left, right, th], axis=1)
    z2 = jnp.dot(ci, cW1[...], preferred_element_type=jnp.float32) + cb1[...]
    z2 = z2 * lax.logistic(z2)
    out[...] = jnp.dot(z2, cW2[...], preferred_element_type=jnp.float32) + cb2[...]


_gctx_tc = pl.pallas_call(
    _gctx_body,
    in_specs=[
        _fullspec((2 * G, H)), _fullspec((2 * G, 8)), _fullspec((H, H)),
        _fullspec((1, H)), _fullspec((G, 1)), _fullspec((2 * H, H)),
        _fullspec((1, H)), _fullspec((H, H)), _fullspec((1, H)),
        _fullspec((3 * H, H)), _fullspec((1, H)), _fullspec((H, H)),
        _fullspec((1, H))],
    out_specs=_fullspec((G, H)),
    out_shape=jax.ShapeDtypeStruct((G, H), jnp.float32),
    grid=(1,),
)


def _final_body(h0, h1, W, b, out):
    h = jnp.concatenate([h0[...], h1[...]], axis=1)
    out[...] = jnp.dot(h, W[...], preferred_element_type=jnp.float32) + b[...]


_final_tc = pl.pallas_call(
    _final_body,
    grid=(GRID,),
    in_specs=[_rowspec(HH), _rowspec(HH), _fullspec((H, 4)), _fullspec((1, 4))],
    out_specs=_rowspec(4),
    out_shape=jax.ShapeDtypeStruct((N, 4), jnp.float32),
)


def _row(v):
    return v.reshape(1, -1).astype(jnp.float32)


def kernel(x, t, linker_batch, linker_graph_ptr, linker_node_type,
           linker_edge_index, left_x, left_edge_index, left_batch, right_x,
           right_edge_index, right_batch, params):
    NF = left_x.shape[0]
    p = params
    fp = p['frag']

    fx = jnp.concatenate([left_x, right_x], axis=0)
    f_src = jnp.concatenate([left_edge_index[0], right_edge_index[0] + NF])
    f_dst = jnp.concatenate([left_edge_index[1], right_edge_index[1] + NF])
    f_batch = jnp.concatenate([left_batch, right_batch + G])
    fs_r, fd_r = _prep_edges(f_src, f_dst)
    ls_r, ld_r = _prep_edges(linker_edge_index[0], linker_edge_index[1])

    deg_l, deg_f = _deg_call(ld_r, fd_r)

    zero_batch = jnp.zeros((N, 1), jnp.int32)
    zero_gctx = jnp.zeros((G, H), jnp.float32)

    fh = _frag_in(fx, fp['in_W'], _row(fp['in_b']))
    for i in range(len(fp['conv_W'])):
        fa = _agg_call(*fh, fs_r, fd_r)
        fh = _conv_tc_noctx(*fh, *fa, deg_f, zero_batch, zero_gctx,
                            fp['conv_W'][i], _row(fp['conv_b'][i]),
                            _row(fp['ln_g'][i]), _row(fp['ln_b'][i]))
    f_batch2 = f_batch.astype(jnp.int32).reshape(N, 1)
    sums, cnt = _pool_tc(*fh, f_batch2)

    gctx = _gctx_tc(sums, cnt, fp['out_W'], _row(fp['out_b']),
                    t.reshape(G, 1).astype(jnp.float32),
                    p['time_W1'], _row(p['time_b1']),
                    p['time_W2'], _row(p['time_b2']),
                    p['cond_W1'], _row(p['cond_b1']),
                    p['cond_W2'], _row(p['cond_b2']))

    lb = linker_batch.astype(jnp.int32).reshape(N, 1)
    nt = linker_node_type.astype(jnp.int32).reshape(N, 1)
    h = _linker_in(x[0], nt, lb, gctx, p['in_W'], _row(p['in_b']))
    for i in range(len(p['conv_W'])):
        a = _agg_call(*h, ls_r, ld_r)
        h = _conv_tc_ctx(*h, *a, deg_l, lb, gctx,
                         p['conv_W'][i], _row(p['conv_b'][i]),
                         _row(p['ln_g'][i]), _row(p['ln_b'][i]))

    out = _final_tc(*h, p['out_W'], _row(p['out_b']))
    return out[None]


# speedup vs baseline: 4.7921x; 1.0532x over previous
import functools
import math

import jax
import jax.numpy as jnp
from jax import lax
from jax.experimental import pallas as pl
from jax.experimental.pallas import tpu as pltpu
from jax.experimental.pallas import tpu_sc as plsc

N = 50000
E = 800000
G = 128
H = 64
HH = 32
NC, NS, LANES = 2, 16, 16

EPT = E // NS
EPT_PAD = 51200
GEDG = 640
NGRP = EPT_PAD // GEDG
NP = 50048
TROWS = NP // NS
CHUNKS = [(k * GEDG, GEDG) for k in range(4)] + [(4 * GEDG, TROWS - 4 * GEDG)]
SRC_PAD = N
DST_PAD = N

BLK = 2000
GRID = N // BLK

_mesh = plsc.VectorSubcoreMesh(core_axis_name="c", subcore_axis_name="s",
                               num_cores=NC, num_subcores=NS)
_CP = pltpu.CompilerParams(use_tc_tiling_on_sc=False)


def _zero_vmem_rows(buf, nrows, width):
    def body(i, _):
        for w0 in range(0, width, LANES):
            buf[i, pl.ds(w0, LANES)] = jnp.zeros((LANES,), jnp.float32)
        return 0
    lax.fori_loop(0, nrows, body, 0)


def _agg_body(h0, h1, src_r, dst_r, o0, o1,
              src_v, dst_v, rows_v, acc, gsem, ssem):
    c = lax.axis_index("c")
    s = lax.axis_index("s")

    def run(h_ref, out_ref):
        _zero_vmem_rows(rows_v, GEDG, HH)
        base = s * TROWS
        for off, ln in CHUNKS:
            pltpu.sync_copy(rows_v.at[pl.ds(0, ln)],
                            acc.at[pl.ds(base + off, ln)])
        plsc.subcore_barrier()

        e0 = s * EPT_PAD

        def grp(g, _):
            r = e0 + g * GEDG
            pltpu.sync_copy(src_r.at[pl.ds(r, GEDG)], src_v)
            pltpu.sync_copy(dst_r.at[pl.ds(r, GEDG)], dst_v)
            pltpu.async_copy(h_ref.at[src_v], rows_v, gsem).wait()
            pltpu.async_copy(rows_v, acc.at[dst_v], ssem, add=True).wait()
            return 0

        lax.fori_loop(0, NGRP, grp, 0)
        plsc.subcore_barrier()

        for off, ln in CHUNKS:
            pltpu.sync_copy(acc.at[pl.ds(base + off, ln)],
                            rows_v.at[pl.ds(0, ln)])
            pltpu.sync_copy(rows_v.at[pl.ds(0, ln)],
                            out_ref.at[pl.ds(base + off, ln)])

    @pl.when(c == 0)
    def _():
        run(h0, o0)

    @pl.when(c == 1)
    def _():
        run(h1, o1)


_agg_call = pl.kernel(
    _agg_body,
    out_type=(jax.ShapeDtypeStruct((NP, HH), jnp.float32),
              jax.ShapeDtypeStruct((NP, HH), jnp.float32)),
    mesh=_mesh,
    compiler_params=_CP,
    scratch_types=[
        pltpu.VMEM((GEDG,), jnp.int32),
        pltpu.VMEM((GEDG,), jnp.int32),
        pltpu.VMEM((GEDG, HH), jnp.float32),
        pltpu.VMEM_SHARED((NP, HH), jnp.float32),
        pltpu.SemaphoreType.DMA,
        pltpu.SemaphoreType.DMA,
    ],
)


def _deg_body(dst_l, dst_f, out_l, out_f, dst_v, ones_v, zbuf, acc, sem):
    c = lax.axis_index("c")
    s = lax.axis_index("s")

    def run(dst_r, out_ref):
        io = lax.broadcasted_iota(jnp.int32, (LANES,), 0)
        one_row = jnp.where(io == 0, 1.0, 0.0).astype(jnp.float32)

        def seto(i, _):
            ones_v[i, pl.ds(0, LANES)] = one_row
            return 0
        lax.fori_loop(0, GEDG, seto, 0)

        _zero_vmem_rows(zbuf, GEDG, LANES)
        base = s * TROWS
        for off, ln in CHUNKS:
            pltpu.sync_copy(zbuf.at[pl.ds(0, ln)],
                            acc.at[pl.ds(base + off, ln)])
        plsc.subcore_barrier()

        e0 = s * EPT_PAD

        def grp(g, _):
            r = e0 + g * GEDG
            pltpu.sync_copy(dst_r.at[pl.ds(r, GEDG)], dst_v)
            pltpu.async_copy(ones_v, acc.at[dst_v], sem, add=True).wait()
            return 0

        lax.fori_loop(0, NGRP, grp, 0)
        plsc.subcore_barrier()

        for off, ln in CHUNKS:
            pltpu.sync_copy(acc.at[pl.ds(base + off, ln)],
                            zbuf.at[pl.ds(0, ln)])
            pltpu.sync_copy(zbuf.at[pl.ds(0, ln)],
                            out_ref.at[pl.ds(base + off, ln)])

    @pl.when(c == 0)
    def _():
        run(dst_l, out_l)

    @pl.when(c == 1)
    def _():
        run(dst_f, out_f)


_deg_call = pl.kernel(
    _deg_body,
    out_type=(jax.ShapeDtypeStruct((NP, LANES), jnp.float32),
              jax.ShapeDtypeStruct((NP, LANES), jnp.float32)),
    mesh=_mesh,
    compiler_params=_CP,
    scratch_types=[
        pltpu.VMEM((GEDG,), jnp.int32),
        pltpu.VMEM((GEDG, LANES), jnp.float32),
        pltpu.VMEM((GEDG, LANES), jnp.float32),
        pltpu.VMEM_SHARED((NP, LANES), jnp.float32),
        pltpu.SemaphoreType.DMA,
    ],
)


def _prep_edges(src, dst):
    s = src.astype(jnp.int32).reshape(NS, EPT)
    d = dst.astype(jnp.int32).reshape(NS, EPT)
    s = jnp.pad(s, ((0, 0), (0, EPT_PAD - EPT)), constant_values=SRC_PAD)
    d = jnp.pad(d, ((0, 0), (0, EPT_PAD - EPT)), constant_values=DST_PAD)
    return s.reshape(NS * EPT_PAD), d.reshape(NS * EPT_PAD)


def _rowspec(w):
    return pl.BlockSpec((BLK, w), lambda i: (i, 0))


def _fullspec(shape):
    return pl.BlockSpec(shape, lambda i: (0,) * len(shape))


def _ln_relu(y, g, b):
    m = jnp.mean(y, axis=-1, keepdims=True)
    v = jnp.mean((y - m) ** 2, axis=-1, keepdims=True)
    return jnp.maximum((y - m) / jnp.sqrt(v + 1e-5) * g + b, 0.0)


def _conv_tc_body(has_ctx, h0, h1, a0, a1, deg, batch, gctx, W, b, g, bb,
                  o0, o1):
    h = jnp.concatenate([h0[...], h1[...]], axis=1)
    a = jnp.concatenate([a0[...], a1[...]], axis=1)
    d = jnp.maximum(deg[...][:, 0:1], 1.0)
    y = jnp.dot(h + a / d, W[...], preferred_element_type=jnp.float32) + b[...]
    if has_ctx:
        io = lax.broadcasted_iota(jnp.int32, (BLK, G), 1)
        oh = (batch[...] == io).astype(jnp.float32)
        y = y + jnp.dot(oh, gctx[...], preferred_element_type=jnp.float32)
    r = _ln_relu(y, g[...], bb[...])
    o0[...] = r[:, :HH]
    o1[...] = r[:, HH:]


def _make_conv_tc(has_ctx):
    in_specs = [_rowspec(HH)] * 4 + [
        _rowspec(LANES), _rowspec(1), _fullspec((G, H)),
        _fullspec((H, H)), _fullspec((1, H)), _fullspec((1, H)),
        _fullspec((1, H))]
    return pl.pallas_call(
        functools.partial(_conv_tc_body, has_ctx),
        grid=(GRID,),
        in_specs=in_specs,
        out_specs=(_rowspec(HH), _rowspec(HH)),
        out_shape=(jax.ShapeDtypeStruct((NP, HH), jnp.float32),
                   jax.ShapeDtypeStruct((NP, HH), jnp.float32)),
    )


_conv_tc_ctx = _make_conv_tc(True)
_conv_tc_noctx = _make_conv_tc(False)


def _frag_in_body(fx, W, b, o0, o1):
    y = jnp.dot(fx[...], W[...], preferred_element_type=jnp.float32) + b[...]
    o0[...] = y[:, :HH]
    o1[...] = y[:, HH:]


_frag_in = pl.pallas_call(
    _frag_in_body,
    grid=(GRID,),
    in_specs=[_rowspec(4), _fullspec((4, H)), _fullspec((1, H))],
    out_specs=(_rowspec(HH), _rowspec(HH)),
    out_shape=(jax.ShapeDtypeStruct((NP, HH), jnp.float32),
               jax.ShapeDtypeStruct((NP, HH), jnp.float32)),
)


def _linker_in_body(x, nt, batch, gctx, W, b, o0, o1):
    xv = x[...]
    ntv = nt[...]
    y = jnp.dot(xv, W[...][:4, :], preferred_element_type=jnp.float32) + b[...]
    ntc = jnp.clip(ntv, 0, 2)
    for k in range(3):
        y = y + (ntc == k).astype(jnp.float32) * W[...][4 + k:5 + k, :]
    y = y + (ntv > 0).astype(jnp.float32) * W[...][7:8, :]
    io = lax.broadcasted_iota(jnp.int32, (BLK, G), 1)
    oh = (batch[...] == io).astype(jnp.float32)
    y = y + jnp.dot(oh, gctx[...], preferred_element_type=jnp.float32)
    o0[...] = y[:, :HH]
    o1[...] = y[:, HH:]


_linker_in = pl.pallas_call(
    _linker_in_body,
    grid=(GRID,),
    in_specs=[_rowspec(4), _rowspec(1), _rowspec(1), _fullspec((G, H)),
              _fullspec((8, H)), _fullspec((1, H))],
    out_specs=(_rowspec(HH), _rowspec(HH)),
    out_shape=(jax.ShapeDtypeStruct((NP, HH), jnp.float32),
               jax.ShapeDtypeStruct((NP, HH), jnp.float32)),
)


def _pool_body(h0, h1, batch, sums, cnt):
    i = pl.program_id(0)

    @pl.when(i == 0)
    def _():
        sums[...] = jnp.zeros_like(sums)
        cnt[...] = jnp.zeros_like(cnt)

    h = jnp.concatenate([h0[...], h1[...]], axis=1)
    io = lax.broadcasted_iota(jnp.int32, (BLK, 2 * G), 1)
    oh = (batch[...] == io).astype(jnp.float32)
    dn = (((0,), (0,)), ((), ()))
    sums[...] += lax.dot_general(oh, h, dn,
                                 preferred_element_type=jnp.float32)
    cnt[...] += lax.dot_general(oh, jnp.ones((BLK, 8), jnp.float32), dn,
                                preferred_element_type=jnp.float32)


_pool_tc = pl.pallas_call(
    _pool_body,
    grid=(GRID,),
    in_specs=[_rowspec(HH), _rowspec(HH), _rowspec(1)],
    out_specs=(_fullspec((2 * G, H)), _fullspec((2 * G, 8))),
    out_shape=(jax.ShapeDtypeStruct((2 * G, H), jnp.float32),
               jax.ShapeDtypeStruct((2 * G, 8), jnp.float32)),
)


def _gctx_body(sums, cnt, fW, fb, t, tW1, tb1, tW2, tb2, cW1, cb1, cW2, cb2,
               out):
    pooled = sums[...] / jnp.maximum(cnt[...][:, 0:1], 1.0)
    fctx = jnp.dot(pooled, fW[...], preferred_element_type=jnp.float32) + fb[...]
    left = fctx[:G, :]
    right = fctx[G:, :]
    j = lax.broadcasted_iota(jnp.int32, (1, H), 1).astype(jnp.float32)
    freqs = jnp.exp(-math.log(10000.0) * j / 64.0)
    a = t[...] * freqs
    te = jnp.concatenate([jnp.sin(a), jnp.cos(a)], axis=1)
    z = jnp.dot(te, tW1[...], preferred_element_type=jnp.float32) + tb1[...]
    z = z * lax.logistic(z)
    th = jnp.dot(z, tW2[...], preferred_element_type=jnp.float32) + tb2[...]
    ci = jnp.concatenate([left, right, th], axis=1)
    z2 = jnp.dot(ci, cW1[...], preferred_element_type=jnp.float32) + cb1[...]
    z2 = z2 * lax.logistic(z2)
    out[...] = jnp.dot(z2, cW2[...], preferred_element_type=jnp.float32) + cb2[...]


_gctx_tc = pl.pallas_call(
    _gctx_body,
    in_specs=[
        _fullspec((2 * G, H)), _fullspec((2 * G, 8)), _fullspec((H, H)),
        _fullspec((1, H)), _fullspec((G, 1)), _fullspec((2 * H, H)),
        _fullspec((1, H)), _fullspec((H, H)), _fullspec((1, H)),
        _fullspec((3 * H, H)), _fullspec((1, H)), _fullspec((H, H)),
        _fullspec((1, H))],
    out_specs=_fullspec((G, H)),
    out_shape=jax.ShapeDtypeStruct((G, H), jnp.float32),
    grid=(1,),
)


def _final_body(h0, h1, W, b, out):
    h = jnp.concatenate([h0[...], h1[...]], axis=1)
    out[...] = jnp.dot(h, W[...], preferred_element_type=jnp.float32) + b[...]


_final_tc = pl.pallas_call(
    _final_body,
    grid=(GRID,),
    in_specs=[_rowspec(HH), _rowspec(HH), _fullspec((H, 4)), _fullspec((1, 4))],
    out_specs=_rowspec(4),
    out_shape=jax.ShapeDtypeStruct((N, 4), jnp.float32),
)


def _row(v):
    return v.reshape(1, -1).astype(jnp.float32)


def kernel(x, t, linker_batch, linker_graph_ptr, linker_node_type,
           linker_edge_index, left_x, left_edge_index, left_batch, right_x,
           right_edge_index, right_batch, params):
    NF = left_x.shape[0]
    p = params
    fp = p['frag']

    fx = jnp.concatenate([left_x, right_x], axis=0)
    f_src = jnp.concatenate([left_edge_index[0], right_edge_index[0] + NF])
    f_dst = jnp.concatenate([left_edge_index[1], right_edge_index[1] + NF])
    f_batch = jnp.concatenate([left_batch, right_batch + G])
    fs_r, fd_r = _prep_edges(f_src, f_dst)
    ls_r, ld_r = _prep_edges(linker_edge_index[0], linker_edge_index[1])

    deg_l, deg_f = _deg_call(ld_r, fd_r)

    zero_batch = jnp.zeros((N, 1), jnp.int32)
    zero_gctx = jnp.zeros((G, H), jnp.float32)

    fh = _frag_in(fx, fp['in_W'], _row(fp['in_b']))
    for i in range(len(fp['conv_W'])):
        fa = _agg_call(*fh, fs_r, fd_r)
        fh = _conv_tc_noctx(*fh, *fa, deg_f, zero_batch, zero_gctx,
                            fp['conv_W'][i], _row(fp['conv_b'][i]),
                            _row(fp['ln_g'][i]), _row(fp['ln_b'][i]))
    f_batch2 = f_batch.astype(jnp.int32).reshape(N, 1)
    sums, cnt = _pool_tc(*fh, f_batch2)

    gctx = _gctx_tc(sums, cnt, fp['out_W'], _row(fp['out_b']),
                    t.reshape(G, 1).astype(jnp.float32),
                    p['time_W1'], _row(p['time_b1']),
                    p['time_W2'], _row(p['time_b2']),
                    p['cond_W1'], _row(p['cond_b1']),
                    p['cond_W2'], _row(p['cond_b2']))

    lb = linker_batch.astype(jnp.int32).reshape(N, 1)
    nt = linker_node_type.astype(jnp.int32).reshape(N, 1)
    h = _linker_in(x[0], nt, lb, gctx, p['in_W'], _row(p['in_b']))
    for i in range(len(p['conv_W'])):
        a = _agg_call(*h, ls_r, ld_r)
        h = _conv_tc_ctx(*h, *a, deg_l, lb, gctx,
                         p['conv_W'][i], _row(p['conv_b'][i]),
                         _row(p['ln_g'][i]), _row(p['ln_b'][i]))

    out = _final_tc(*h, p['out_W'], _row(p['out_b']))
    return out[None]


# speedup vs baseline: 4.8497x; 1.0120x over previous
import functools
import math

import jax
import jax.numpy as jnp
from jax import lax
from jax.experimental import pallas as pl
from jax.experimental.pallas import tpu as pltpu
from jax.experimental.pallas import tpu_sc as plsc

N = 50000
E = 800000
G = 128
H = 64
HH = 32
NC, NS, LANES = 2, 16, 16

EPT = E // NS
EPT_PAD = 51200
GEDG = 320
NGRP = EPT_PAD // GEDG
NP = 50048
TROWS = NP // NS
CHUNKS = [(k * GEDG, GEDG) for k in range(9)] + [(9 * GEDG, TROWS - 9 * GEDG)]
SRC_PAD = N
DST_PAD = N

BLK = 2000
GRID = N // BLK

_mesh = plsc.VectorSubcoreMesh(core_axis_name="c", subcore_axis_name="s",
                               num_cores=NC, num_subcores=NS)
_CP = pltpu.CompilerParams(use_tc_tiling_on_sc=False)


def _zero_vmem_rows(buf, nrows, width):
    def body(i, _):
        for w0 in range(0, width, LANES):
            buf[i, pl.ds(w0, LANES)] = jnp.zeros((LANES,), jnp.float32)
        return 0
    lax.fori_loop(0, nrows, body, 0)


def _agg_body(h0, h1, src_r, dst_r, o0, o1,
              src_v, dst_v, rows_v, acc, gsem, ssem):
    c = lax.axis_index("c")
    s = lax.axis_index("s")

    def run(h_ref, out_ref):
        _zero_vmem_rows(rows_v.at[0], GEDG, HH)
        base = s * TROWS
        for off, ln in CHUNKS:
            pltpu.sync_copy(rows_v.at[0].at[pl.ds(0, ln)],
                            acc.at[pl.ds(base + off, ln)])
        plsc.subcore_barrier()

        e0 = s * EPT_PAD

        def idx_load(g, p):
            r = e0 + g * GEDG
            pltpu.sync_copy(src_r.at[pl.ds(r, GEDG)], src_v.at[p])
            pltpu.sync_copy(dst_r.at[pl.ds(r, GEDG)], dst_v.at[p])

        def gather(p):
            return pltpu.async_copy(h_ref.at[src_v.at[p]], rows_v.at[p], gsem)

        def scatter(p):
            return pltpu.async_copy(rows_v.at[p], acc.at[dst_v.at[p]], ssem,
                                    add=True)

        def drain_scatter(p):
            pltpu.make_async_copy(rows_v.at[p], acc.at[dst_v.at[p]],
                                  ssem).wait()

        idx_load(0, 0)
        g0 = gather(0)
        idx_load(1, 1)
        g0.wait()
        scatter(0)
        g1 = gather(1)
        g1.wait()
        scatter(1)

        def step(gi, _):
            ga = 2 * gi
            idx_load(ga, 0)
            drain_scatter(0)
            da = gather(0)
            idx_load(ga + 1, 1)
            da.wait()
            scatter(0)
            drain_scatter(1)
            db = gather(1)
            db.wait()
            scatter(1)
            return 0

        lax.fori_loop(1, NGRP // 2, step, 0)
        drain_scatter(0)
        drain_scatter(1)
        plsc.subcore_barrier()

        for off, ln in CHUNKS:
            pltpu.sync_copy(acc.at[pl.ds(base + off, ln)],
                            rows_v.at[0].at[pl.ds(0, ln)])
            pltpu.sync_copy(rows_v.at[0].at[pl.ds(0, ln)],
                            out_ref.at[pl.ds(base + off, ln)])

    @pl.when(c == 0)
    def _():
        run(h0, o0)

    @pl.when(c == 1)
    def _():
        run(h1, o1)


_agg_call = pl.kernel(
    _agg_body,
    out_type=(jax.ShapeDtypeStruct((NP, HH), jnp.float32),
              jax.ShapeDtypeStruct((NP, HH), jnp.float32)),
    mesh=_mesh,
    compiler_params=_CP,
    scratch_types=[
        pltpu.VMEM((2, GEDG), jnp.int32),
        pltpu.VMEM((2, GEDG), jnp.int32),
        pltpu.VMEM((2, GEDG, HH), jnp.float32),
        pltpu.VMEM_SHARED((NP, HH), jnp.float32),
        pltpu.SemaphoreType.DMA,
        pltpu.SemaphoreType.DMA,
    ],
)


def _deg_body(dst_l, dst_f, out_l, out_f, dst_v, ones_v, zbuf, acc, sem):
    c = lax.axis_index("c")
    s = lax.axis_index("s")

    def run(dst_r, out_ref):
        io = lax.broadcasted_iota(jnp.int32, (LANES,), 0)
        one_row = jnp.where(io == 0, 1.0, 0.0).astype(jnp.float32)

        def seto(i, _):
            ones_v[i, pl.ds(0, LANES)] = one_row
            return 0
        lax.fori_loop(0, GEDG, seto, 0)

        _zero_vmem_rows(zbuf, GEDG, LANES)
        base = s * TROWS
        for off, ln in CHUNKS:
            pltpu.sync_copy(zbuf.at[pl.ds(0, ln)],
                            acc.at[pl.ds(base + off, ln)])
        plsc.subcore_barrier()

        e0 = s * EPT_PAD

        def grp(g, _):
            r = e0 + g * GEDG
            pltpu.sync_copy(dst_r.at[pl.ds(r, GEDG)], dst_v)
            pltpu.async_copy(ones_v, acc.at[dst_v], sem, add=True).wait()
            return 0

        lax.fori_loop(0, NGRP, grp, 0)
        plsc.subcore_barrier()

        for off, ln in CHUNKS:
            pltpu.sync_copy(acc.at[pl.ds(base + off, ln)],
                            zbuf.at[pl.ds(0, ln)])
            pltpu.sync_copy(zbuf.at[pl.ds(0, ln)],
                            out_ref.at[pl.ds(base + off, ln)])

    @pl.when(c == 0)
    def _():
        run(dst_l, out_l)

    @pl.when(c == 1)
    def _():
        run(dst_f, out_f)


_deg_call = pl.kernel(
    _deg_body,
    out_type=(jax.ShapeDtypeStruct((NP, LANES), jnp.float32),
              jax.ShapeDtypeStruct((NP, LANES), jnp.float32)),
    mesh=_mesh,
    compiler_params=_CP,
    scratch_types=[
        pltpu.VMEM((GEDG,), jnp.int32),
        pltpu.VMEM((GEDG, LANES), jnp.float32),
        pltpu.VMEM((GEDG, LANES), jnp.float32),
        pltpu.VMEM_SHARED((NP, LANES), jnp.float32),
        pltpu.SemaphoreType.DMA,
    ],
)


def _prep_edges(src, dst):
    s = src.astype(jnp.int32).reshape(NS, EPT)
    d = dst.astype(jnp.int32).reshape(NS, EPT)
    s = jnp.pad(s, ((0, 0), (0, EPT_PAD - EPT)), constant_values=SRC_PAD)
    d = jnp.pad(d, ((0, 0), (0, EPT_PAD - EPT)), constant_values=DST_PAD)
    return s.reshape(NS * EPT_PAD), d.reshape(NS * EPT_PAD)


def _rowspec(w):
    return pl.BlockSpec((BLK, w), lambda i: (i, 0))


def _fullspec(shape):
    return pl.BlockSpec(shape, lambda i: (0,) * len(shape))


def _ln_relu(y, g, b):
    m = jnp.mean(y, axis=-1, keepdims=True)
    v = jnp.mean((y - m) ** 2, axis=-1, keepdims=True)
    return jnp.maximum((y - m) / jnp.sqrt(v + 1e-5) * g + b, 0.0)


def _conv_tc_body(has_ctx, h0, h1, a0, a1, deg, batch, gctx, W, b, g, bb,
                  o0, o1):
    h = jnp.concatenate([h0[...], h1[...]], axis=1)
    a = jnp.concatenate([a0[...], a1[...]], axis=1)
    d = jnp.maximum(deg[...][:, 0:1], 1.0)
    y = jnp.dot(h + a / d, W[...], preferred_element_type=jnp.float32) + b[...]
    if has_ctx:
        io = lax.broadcasted_iota(jnp.int32, (BLK, G), 1)
        oh = (batch[...] == io).astype(jnp.float32)
        y = y + jnp.dot(oh, gctx[...], preferred_element_type=jnp.float32)
    r = _ln_relu(y, g[...], bb[...])
    o0[...] = r[:, :HH]
    o1[...] = r[:, HH:]


def _make_conv_tc(has_ctx):
    in_specs = [_rowspec(HH)] * 4 + [
        _rowspec(LANES), _rowspec(1), _fullspec((G, H)),
        _fullspec((H, H)), _fullspec((1, H)), _fullspec((1, H)),
        _fullspec((1, H))]
    return pl.pallas_call(
        functools.partial(_conv_tc_body, has_ctx),
        grid=(GRID,),
        in_specs=in_specs,
        out_specs=(_rowspec(HH), _rowspec(HH)),
        out_shape=(jax.ShapeDtypeStruct((NP, HH), jnp.float32),
                   jax.ShapeDtypeStruct((NP, HH), jnp.float32)),
    )


_conv_tc_ctx = _make_conv_tc(True)
_conv_tc_noctx = _make_conv_tc(False)


def _frag_in_body(fx, W, b, o0, o1):
    y = jnp.dot(fx[...], W[...], preferred_element_type=jnp.float32) + b[...]
    o0[...] = y[:, :HH]
    o1[...] = y[:, HH:]


_frag_in = pl.pallas_call(
    _frag_in_body,
    grid=(GRID,),
    in_specs=[_rowspec(4), _fullspec((4, H)), _fullspec((1, H))],
    out_specs=(_rowspec(HH), _rowspec(HH)),
    out_shape=(jax.ShapeDtypeStruct((NP, HH), jnp.float32),
               jax.ShapeDtypeStruct((NP, HH), jnp.float32)),
)


def _linker_in_body(x, nt, batch, gctx, W, b, o0, o1):
    xv = x[...]
    ntv = nt[...]
    y = jnp.dot(xv, W[...][:4, :], preferred_element_type=jnp.float32) + b[...]
    ntc = jnp.clip(ntv, 0, 2)
    for k in range(3):
        y = y + (ntc == k).astype(jnp.float32) * W[...][4 + k:5 + k, :]
    y = y + (ntv > 0).astype(jnp.float32) * W[...][7:8, :]
    io = lax.broadcasted_iota(jnp.int32, (BLK, G), 1)
    oh = (batch[...] == io).astype(jnp.float32)
    y = y + jnp.dot(oh, gctx[...], preferred_element_type=jnp.float32)
    o0[...] = y[:, :HH]
    o1[...] = y[:, HH:]


_linker_in = pl.pallas_call(
    _linker_in_body,
    grid=(GRID,),
    in_specs=[_rowspec(4), _rowspec(1), _rowspec(1), _fullspec((G, H)),
              _fullspec((8, H)), _fullspec((1, H))],
    out_specs=(_rowspec(HH), _rowspec(HH)),
    out_shape=(jax.ShapeDtypeStruct((NP, HH), jnp.float32),
               jax.ShapeDtypeStruct((NP, HH), jnp.float32)),
)


def _pool_body(h0, h1, batch, sums, cnt):
    i = pl.program_id(0)

    @pl.when(i == 0)
    def _():
        sums[...] = jnp.zeros_like(sums)
        cnt[...] = jnp.zeros_like(cnt)

    h = jnp.concatenate([h0[...], h1[...]], axis=1)
    io = lax.broadcasted_iota(jnp.int32, (BLK, 2 * G), 1)
    oh = (batch[...] == io).astype(jnp.float32)
    dn = (((0,), (0,)), ((), ()))
    sums[...] += lax.dot_general(oh, h, dn,
                                 preferred_element_type=jnp.float32)
    cnt[...] += lax.dot_general(oh, jnp.ones((BLK, 8), jnp.float32), dn,
                                preferred_element_type=jnp.float32)


_pool_tc = pl.pallas_call(
    _pool_body,
    grid=(GRID,),
    in_specs=[_rowspec(HH), _rowspec(HH), _rowspec(1)],
    out_specs=(_fullspec((2 * G, H)), _fullspec((2 * G, 8))),
    out_shape=(jax.ShapeDtypeStruct((2 * G, H), jnp.float32),
               jax.ShapeDtypeStruct((2 * G, 8), jnp.float32)),
)


def _gctx_body(sums, cnt, fW, fb, t, tW1, tb1, tW2, tb2, cW1, cb1, cW2, cb2,
               out):
    pooled = sums[...] / jnp.maximum(cnt[...][:, 0:1], 1.0)
    fctx = jnp.dot(pooled, fW[...], preferred_element_type=jnp.float32) + fb[...]
    left = fctx[:G, :]
    right = fctx[G:, :]
    j = lax.broadcasted_iota(jnp.int32, (1, H), 1).astype(jnp.float32)
    freqs = jnp.exp(-math.log(10000.0) * j / 64.0)
    a = t[...] * freqs
    te = jnp.concatenate([jnp.sin(a), jnp.cos(a)], axis=1)
    z = jnp.dot(te, tW1[...], preferred_element_type=jnp.float32) + tb1[...]
    z = z * lax.logistic(z)
    th = jnp.dot(z, tW2[...], preferred_element_type=jnp.float32) + tb2[...]
    ci = jnp.concatenate([left, right, th], axis=1)
    z2 = jnp.dot(ci, cW1[...], preferred_element_type=jnp.float32) + cb1[...]
    z2 = z2 * lax.logistic(z2)
    out[...] = jnp.dot(z2, cW2[...], preferred_element_type=jnp.float32) + cb2[...]


_gctx_tc = pl.pallas_call(
    _gctx_body,
    in_specs=[
        _fullspec((2 * G, H)), _fullspec((2 * G, 8)), _fullspec((H, H)),
        _fullspec((1, H)), _fullspec((G, 1)), _fullspec((2 * H, H)),
        _fullspec((1, H)), _fullspec((H, H)), _fullspec((1, H)),
        _fullspec((3 * H, H)), _fullspec((1, H)), _fullspec((H, H)),
        _fullspec((1, H))],
    out_specs=_fullspec((G, H)),
    out_shape=jax.ShapeDtypeStruct((G, H), jnp.float32),
    grid=(1,),
)


def _final_body(h0, h1, W, b, out):
    h = jnp.concatenate([h0[...], h1[...]], axis=1)
    out[...] = jnp.dot(h, W[...], preferred_element_type=jnp.float32) + b[...]


_final_tc = pl.pallas_call(
    _final_body,
    grid=(GRID,),
    in_specs=[_rowspec(HH), _rowspec(HH), _fullspec((H, 4)), _fullspec((1, 4))],
    out_specs=_rowspec(4),
    out_shape=jax.ShapeDtypeStruct((N, 4), jnp.float32),
)


def _row(v):
    return v.reshape(1, -1).astype(jnp.float32)


def kernel(x, t, linker_batch, linker_graph_ptr, linker_node_type,
           linker_edge_index, left_x, left_edge_index, left_batch, right_x,
           right_edge_index, right_batch, params):
    NF = left_x.shape[0]
    p = params
    fp = p['frag']

    fx = jnp.concatenate([left_x, right_x], axis=0)
    f_src = jnp.concatenate([left_edge_index[0], right_edge_index[0] + NF])
    f_dst = jnp.concatenate([left_edge_index[1], right_edge_index[1] + NF])
    f_batch = jnp.concatenate([left_batch, right_batch + G])
    fs_r, fd_r = _prep_edges(f_src, f_dst)
    ls_r, ld_r = _prep_edges(linker_edge_index[0], linker_edge_index[1])

    deg_l, deg_f = _deg_call(ld_r, fd_r)

    zero_batch = jnp.zeros((N, 1), jnp.int32)
    zero_gctx = jnp.zeros((G, H), jnp.float32)

    fh = _frag_in(fx, fp['in_W'], _row(fp['in_b']))
    for i in range(len(fp['conv_W'])):
        fa = _agg_call(*fh, fs_r, fd_r)
        fh = _conv_tc_noctx(*fh, *fa, deg_f, zero_batch, zero_gctx,
                            fp['conv_W'][i], _row(fp['conv_b'][i]),
                            _row(fp['ln_g'][i]), _row(fp['ln_b'][i]))
    f_batch2 = f_batch.astype(jnp.int32).reshape(N, 1)
    sums, cnt = _pool_tc(*fh, f_batch2)

    gctx = _gctx_tc(sums, cnt, fp['out_W'], _row(fp['out_b']),
                    t.reshape(G, 1).astype(jnp.float32),
                    p['time_W1'], _row(p['time_b1']),
                    p['time_W2'], _row(p['time_b2']),
                    p['cond_W1'], _row(p['cond_b1']),
                    p['cond_W2'], _row(p['cond_b2']))

    lb = linker_batch.astype(jnp.int32).reshape(N, 1)
    nt = linker_node_type.astype(jnp.int32).reshape(N, 1)
    h = _linker_in(x[0], nt, lb, gctx, p['in_W'], _row(p['in_b']))
    for i in range(len(p['conv_W'])):
        a = _agg_call(*h, ls_r, ld_r)
        h = _conv_tc_ctx(*h, *a, deg_l, lb, gctx,
                         p['conv_W'][i], _row(p['conv_b'][i]),
                         _row(p['ln_g'][i]), _row(p['ln_b'][i]))

    out = _final_tc(*h, p['out_W'], _row(p['out_b']))
    return out[None]


# speedup vs baseline: 5.4083x; 1.1152x over previous
import functools
import math

import jax
import jax.numpy as jnp
from jax import lax
from jax.experimental import pallas as pl
from jax.experimental.pallas import tpu as pltpu
from jax.experimental.pallas import tpu_sc as plsc

N = 50000
E = 800000
G = 128
H = 64
HH = 32
NC, NS, LANES = 2, 16, 16

EPT = E // NS
EPT_PAD = 51200
GEDG = 320
NGRP = EPT_PAD // GEDG
GEDG_D = 1600
NGRP_D = EPT_PAD // GEDG_D
NP = 50048
TROWS = NP // NS
CHUNKS = [(k * GEDG, GEDG) for k in range(9)] + [(9 * GEDG, TROWS - 9 * GEDG)]
SRC_PAD = N
DST_PAD = N

BLK = 2000
GRID = N // BLK

_mesh = plsc.VectorSubcoreMesh(core_axis_name="c", subcore_axis_name="s",
                               num_cores=NC, num_subcores=NS)
_CP = pltpu.CompilerParams(use_tc_tiling_on_sc=False)


def _zero_vmem_rows(buf, nrows, width):
    def body(i, _):
        for w0 in range(0, width, LANES):
            buf[i, pl.ds(w0, LANES)] = jnp.zeros((LANES,), jnp.float32)
        return 0
    lax.fori_loop(0, nrows, body, 0)


def _agg_body(h0, h1, idx_r, o0, o1,
              is0, is1, is2, is3, rows0, rows1, acc,
              isem0, isem1, isem2, isem3, gsem0, gsem1, ssem0, ssem1):
    c = lax.axis_index("c")
    s = lax.axis_index("s")
    islot = (is0, is1, is2, is3)
    isem = (isem0, isem1, isem2, isem3)
    rows = (rows0, rows1)
    gsem = (gsem0, gsem1)
    ssem = (ssem0, ssem1)

    def run(h_ref, out_ref):
        _zero_vmem_rows(rows0, GEDG, HH)
        base = s * TROWS
        for off, ln in CHUNKS:
            pltpu.sync_copy(rows0.at[pl.ds(0, ln)],
                            acc.at[pl.ds(base + off, ln)])
        plsc.subcore_barrier()

        gbase = s * NGRP

        def icopy(g, q):
            pltpu.async_copy(idx_r.at[gbase + g], islot[q], isem[q])

        def idrain(q):
            pltpu.make_async_copy(idx_r.at[gbase], islot[q], isem[q]).wait()

        def gather(q, p):
            pltpu.async_copy(h_ref.at[islot[q].at[0]], rows[p], gsem[p])

        def gwait(p):
            pltpu.make_async_copy(h_ref.at[islot[0].at[0]], rows[p],
                                  gsem[p]).wait()

        def scatter(q, p):
            pltpu.async_copy(rows[p], acc.at[islot[q].at[1]], ssem[p],
                             add=True)

        def sdrain(p):
            pltpu.make_async_copy(rows[p], acc.at[islot[0].at[1]],
                                  ssem[p]).wait()

        def quad(g0, first):
            idrain(0)
            if not first:
                sdrain(0)
            gather(0, 0)
            icopy(g0 + 2, 2)
            idrain(1)
            if not first:
                sdrain(1)
            gather(1, 1)
            icopy(g0 + 3, 3)
            gwait(0)
            scatter(0, 0)
            gwait(1)
            scatter(1, 1)
            idrain(2)
            sdrain(0)
            gather(2, 0)
            icopy(g0 + 4, 0)
            idrain(3)
            sdrain(1)
            gather(3, 1)
            icopy(g0 + 5, 1)
            gwait(0)
            scatter(2, 0)
            gwait(1)
            scatter(3, 1)

        icopy(0, 0)
        icopy(1, 1)
        quad(0, True)

        def step(hb, _):
            quad(4 * hb, False)
            return 0

        lax.fori_loop(1, NGRP // 4, step, 0)
        idrain(0)
        idrain(1)
        sdrain(0)
        sdrain(1)
        plsc.subcore_barrier()

        for off, ln in CHUNKS:
            pltpu.sync_copy(acc.at[pl.ds(base + off, ln)],
                            rows0.at[pl.ds(0, ln)])
            pltpu.sync_copy(rows0.at[pl.ds(0, ln)],
                            out_ref.at[pl.ds(base + off, ln)])

    @pl.when(c == 0)
    def _():
        run(h0, o0)

    @pl.when(c == 1)
    def _():
        run(h1, o1)


_agg_call = pl.kernel(
    _agg_body,
    out_type=(jax.ShapeDtypeStruct((NP, HH), jnp.float32),
              jax.ShapeDtypeStruct((NP, HH), jnp.float32)),
    mesh=_mesh,
    compiler_params=_CP,
    scratch_types=[
        pltpu.VMEM((2, GEDG), jnp.int32),
        pltpu.VMEM((2, GEDG), jnp.int32),
        pltpu.VMEM((2, GEDG), jnp.int32),
        pltpu.VMEM((2, GEDG), jnp.int32),
        pltpu.VMEM((GEDG, HH), jnp.float32),
        pltpu.VMEM((GEDG, HH), jnp.float32),
        pltpu.VMEM_SHARED((NP, HH), jnp.float32),
        pltpu.SemaphoreType.DMA,
        pltpu.SemaphoreType.DMA,
        pltpu.SemaphoreType.DMA,
        pltpu.SemaphoreType.DMA,
        pltpu.SemaphoreType.DMA,
        pltpu.SemaphoreType.DMA,
        pltpu.SemaphoreType.DMA,
        pltpu.SemaphoreType.DMA,
    ],
)


def _deg_body(dst_l, dst_f, out_l, out_f, dv0, dv1, ones_v, zbuf, acc,
              sem0, sem1):
    c = lax.axis_index("c")
    s = lax.axis_index("s")

    def run(dst_r, out_ref):
        io = lax.broadcasted_iota(jnp.int32, (LANES,), 0)
        one_row = jnp.where(io == 0, 1.0, 0.0).astype(jnp.float32)

        def seto(i, _):
            ones_v[i, pl.ds(0, LANES)] = one_row
            return 0
        lax.fori_loop(0, GEDG_D, seto, 0)

        _zero_vmem_rows(zbuf, GEDG, LANES)
        base = s * TROWS
        for off, ln in CHUNKS:
            pltpu.sync_copy(zbuf.at[pl.ds(0, ln)],
                            acc.at[pl.ds(base + off, ln)])
        plsc.subcore_barrier()

        e0 = s * EPT_PAD
        dslot = (dv0, dv1)
        ssem = (sem0, sem1)

        def load(g, q):
            pltpu.sync_copy(dst_r.at[pl.ds(e0 + g * GEDG_D, GEDG_D)],
                            dslot[q])

        def scatter(q):
            pltpu.async_copy(ones_v, acc.at[dslot[q]], ssem[q], add=True)

        def sdrain(q):
            pltpu.make_async_copy(ones_v, acc.at[dslot[q]], ssem[q]).wait()

        load(0, 0)
        scatter(0)
        load(1, 1)
        scatter(1)

        def step(hb, _):
            sdrain(0)
            load(2 * hb, 0)
            scatter(0)
            sdrain(1)
            load(2 * hb + 1, 1)
            scatter(1)
            return 0

        lax.fori_loop(1, NGRP_D // 2, step, 0)
        sdrain(0)
        sdrain(1)
        plsc.subcore_barrier()

        for off, ln in CHUNKS:
            pltpu.sync_copy(acc.at[pl.ds(base + off, ln)],
                            zbuf.at[pl.ds(0, ln)])
            pltpu.sync_copy(zbuf.at[pl.ds(0, ln)],
                            out_ref.at[pl.ds(base + off, ln)])

    @pl.when(c == 0)
    def _():
        run(dst_l, out_l)

    @pl.when(c == 1)
    def _():
        run(dst_f, out_f)


_deg_call = pl.kernel(
    _deg_body,
    out_type=(jax.ShapeDtypeStruct((NP, LANES), jnp.float32),
              jax.ShapeDtypeStruct((NP, LANES), jnp.float32)),
    mesh=_mesh,
    compiler_params=_CP,
    scratch_types=[
        pltpu.VMEM((GEDG_D,), jnp.int32),
        pltpu.VMEM((GEDG_D,), jnp.int32),
        pltpu.VMEM((GEDG_D, LANES), jnp.float32),
        pltpu.VMEM((GEDG, LANES), jnp.float32),
        pltpu.VMEM_SHARED((NP, LANES), jnp.float32),
        pltpu.SemaphoreType.DMA,
        pltpu.SemaphoreType.DMA,
    ],
)


def _prep_edges(src, dst):
    s = src.astype(jnp.int32).reshape(NS, EPT)
    d = dst.astype(jnp.int32).reshape(NS, EPT)
    s = jnp.pad(s, ((0, 0), (0, EPT_PAD - EPT)), constant_values=SRC_PAD)
    d = jnp.pad(d, ((0, 0), (0, EPT_PAD - EPT)), constant_values=DST_PAD)
    packed = jnp.stack([s.reshape(NS, NGRP, GEDG),
                        d.reshape(NS, NGRP, GEDG)], axis=2)
    packed = packed.reshape(NS * NGRP, 2, GEDG)
    packed = jnp.pad(packed, ((0, 2), (0, 0), (0, 0)))
    return packed, d.reshape(NS * EPT_PAD)


def _rowspec(w):
    return pl.BlockSpec((BLK, w), lambda i: (i, 0))


def _fullspec(shape):
    return pl.BlockSpec(shape, lambda i: (0,) * len(shape))


def _ln_relu(y, g, b):
    m = jnp.mean(y, axis=-1, keepdims=True)
    v = jnp.mean((y - m) ** 2, axis=-1, keepdims=True)
    return jnp.maximum((y - m) / jnp.sqrt(v + 1e-5) * g + b, 0.0)


def _conv_tc_body(has_ctx, h0, h1, a0, a1, deg, batch, gctx, W, b, g, bb,
                  o0, o1):
    h = jnp.concatenate([h0[...], h1[...]], axis=1)
    a = jnp.concatenate([a0[...], a1[...]], axis=1)
    d = jnp.maximum(deg[...][:, 0:1], 1.0)
    y = jnp.dot(h + a / d, W[...], preferred_element_type=jnp.float32) + b[...]
    if has_ctx:
        io = lax.broadcasted_iota(jnp.int32, (BLK, G), 1)
        oh = (batch[...] == io).astype(jnp.float32)
        y = y + jnp.dot(oh, gctx[...], preferred_element_type=jnp.float32)
    r = _ln_relu(y, g[...], bb[...])
    o0[...] = r[:, :HH]
    o1[...] = r[:, HH:]


def _make_conv_tc(has_ctx):
    in_specs = [_rowspec(HH)] * 4 + [
        _rowspec(LANES), _rowspec(1), _fullspec((G, H)),
        _fullspec((H, H)), _fullspec((1, H)), _fullspec((1, H)),
        _fullspec((1, H))]
    return pl.pallas_call(
        functools.partial(_conv_tc_body, has_ctx),
        grid=(GRID,),
        in_specs=in_specs,
        out_specs=(_rowspec(HH), _rowspec(HH)),
        out_shape=(jax.ShapeDtypeStruct((NP, HH), jnp.float32),
                   jax.ShapeDtypeStruct((NP, HH), jnp.float32)),
    )


_conv_tc_ctx = _make_conv_tc(True)
_conv_tc_noctx = _make_conv_tc(False)


def _frag_in_body(fx, W, b, o0, o1):
    y = jnp.dot(fx[...], W[...], preferred_element_type=jnp.float32) + b[...]
    o0[...] = y[:, :HH]
    o1[...] = y[:, HH:]


_frag_in = pl.pallas_call(
    _frag_in_body,
    grid=(GRID,),
    in_specs=[_rowspec(4), _fullspec((4, H)), _fullspec((1, H))],
    out_specs=(_rowspec(HH), _rowspec(HH)),
    out_shape=(jax.ShapeDtypeStruct((NP, HH), jnp.float32),
               jax.ShapeDtypeStruct((NP, HH), jnp.float32)),
)


def _linker_in_body(x, nt, batch, gctx, W, b, o0, o1):
    xv = x[...]
    ntv = nt[...]
    y = jnp.dot(xv, W[...][:4, :], preferred_element_type=jnp.float32) + b[...]
    ntc = jnp.clip(ntv, 0, 2)
    for k in range(3):
        y = y + (ntc == k).astype(jnp.float32) * W[...][4 + k:5 + k, :]
    y = y + (ntv > 0).astype(jnp.float32) * W[...][7:8, :]
    io = lax.broadcasted_iota(jnp.int32, (BLK, G), 1)
    oh = (batch[...] == io).astype(jnp.float32)
    y = y + jnp.dot(oh, gctx[...], preferred_element_type=jnp.float32)
    o0[...] = y[:, :HH]
    o1[...] = y[:, HH:]


_linker_in = pl.pallas_call(
    _linker_in_body,
    grid=(GRID,),
    in_specs=[_rowspec(4), _rowspec(1), _rowspec(1), _fullspec((G, H)),
              _fullspec((8, H)), _fullspec((1, H))],
    out_specs=(_rowspec(HH), _rowspec(HH)),
    out_shape=(jax.ShapeDtypeStruct((NP, HH), jnp.float32),
               jax.ShapeDtypeStruct((NP, HH), jnp.float32)),
)


def _pool_body(h0, h1, batch, sums, cnt):
    i = pl.program_id(0)

    @pl.when(i == 0)
    def _():
        sums[...] = jnp.zeros_like(sums)
        cnt[...] = jnp.zeros_like(cnt)

    h = jnp.concatenate([h0[...], h1[...]], axis=1)
    io = lax.broadcasted_iota(jnp.int32, (BLK, 2 * G), 1)
    oh = (batch[...] == io).astype(jnp.float32)
    dn = (((0,), (0,)), ((), ()))
    sums[...] += lax.dot_general(oh, h, dn,
                                 preferred_element_type=jnp.float32)
    cnt[...] += lax.dot_general(oh, jnp.ones((BLK, 8), jnp.float32), dn,
                                preferred_element_type=jnp.float32)


_pool_tc = pl.pallas_call(
    _pool_body,
    grid=(GRID,),
    in_specs=[_rowspec(HH), _rowspec(HH), _rowspec(1)],
    out_specs=(_fullspec((2 * G, H)), _fullspec((2 * G, 8))),
    out_shape=(jax.ShapeDtypeStruct((2 * G, H), jnp.float32),
               jax.ShapeDtypeStruct((2 * G, 8), jnp.float32)),
)


def _gctx_body(sums, cnt, fW, fb, t, tW1, tb1, tW2, tb2, cW1, cb1, cW2, cb2,
               out):
    pooled = sums[...] / jnp.maximum(cnt[...][:, 0:1], 1.0)
    fctx = jnp.dot(pooled, fW[...], preferred_element_type=jnp.float32) + fb[...]
    left = fctx[:G, :]
    right = fctx[G:, :]
    j = lax.broadcasted_iota(jnp.int32, (1, H), 1).astype(jnp.float32)
    freqs = jnp.exp(-math.log(10000.0) * j / 64.0)
    a = t[...] * freqs
    te = jnp.concatenate([jnp.sin(a), jnp.cos(a)], axis=1)
    z = jnp.dot(te, tW1[...], preferred_element_type=jnp.float32) + tb1[...]
    z = z * lax.logistic(z)
    th = jnp.dot(z, tW2[...], preferred_element_type=jnp.float32) + tb2[...]
    ci = jnp.concatenate([left, right, th], axis=1)
    z2 = jnp.dot(ci, cW1[...], preferred_element_type=jnp.float32) + cb1[...]
    z2 = z2 * lax.logistic(z2)
    out[...] = jnp.dot(z2, cW2[...], preferred_element_type=jnp.float32) + cb2[...]


_gctx_tc = pl.pallas_call(
    _gctx_body,
    in_specs=[
        _fullspec((2 * G, H)), _fullspec((2 * G, 8)), _fullspec((H, H)),
        _fullspec((1, H)), _fullspec((G, 1)), _fullspec((2 * H, H)),
        _fullspec((1, H)), _fullspec((H, H)), _fullspec((1, H)),
        _fullspec((3 * H, H)), _fullspec((1, H)), _fullspec((H, H)),
        _fullspec((1, H))],
    out_specs=_fullspec((G, H)),
    out_shape=jax.ShapeDtypeStruct((G, H), jnp.float32),
    grid=(1,),
)


def _final_body(h0, h1, W, b, out):
    h = jnp.concatenate([h0[...], h1[...]], axis=1)
    out[...] = jnp.dot(h, W[...], preferred_element_type=jnp.float32) + b[...]


_final_tc = pl.pallas_call(
    _final_body,
    grid=(GRID,),
    in_specs=[_rowspec(HH), _rowspec(HH), _fullspec((H, 4)), _fullspec((1, 4))],
    out_specs=_rowspec(4),
    out_shape=jax.ShapeDtypeStruct((N, 4), jnp.float32),
)


def _row(v):
    return v.reshape(1, -1).astype(jnp.float32)


def kernel(x, t, linker_batch, linker_graph_ptr, linker_node_type,
           linker_edge_index, left_x, left_edge_index, left_batch, right_x,
           right_edge_index, right_batch, params):
    NF = left_x.shape[0]
    p = params
    fp = p['frag']

    fx = jnp.concatenate([left_x, right_x], axis=0)
    f_src = jnp.concatenate([left_edge_index[0], right_edge_index[0] + NF])
    f_dst = jnp.concatenate([left_edge_index[1], right_edge_index[1] + NF])
    f_batch = jnp.concatenate([left_batch, right_batch + G])
    f_idx, f_dflat = _prep_edges(f_src, f_dst)
    l_idx, l_dflat = _prep_edges(linker_edge_index[0], linker_edge_index[1])

    deg_l, deg_f = _deg_call(l_dflat, f_dflat)

    zero_batch = jnp.zeros((N, 1), jnp.int32)
    zero_gctx = jnp.zeros((G, H), jnp.float32)

    fh = _frag_in(fx, fp['in_W'], _row(fp['in_b']))
    for i in range(len(fp['conv_W'])):
        fa = _agg_call(*fh, f_idx)
        fh = _conv_tc_noctx(*fh, *fa, deg_f, zero_batch, zero_gctx,
                            fp['conv_W'][i], _row(fp['conv_b'][i]),
                            _row(fp['ln_g'][i]), _row(fp['ln_b'][i]))
    f_batch2 = f_batch.astype(jnp.int32).reshape(N, 1)
    sums, cnt = _pool_tc(*fh, f_batch2)

    gctx = _gctx_tc(sums, cnt, fp['out_W'], _row(fp['out_b']),
                    t.reshape(G, 1).astype(jnp.float32),
                    p['time_W1'], _row(p['time_b1']),
                    p['time_W2'], _row(p['time_b2']),
                    p['cond_W1'], _row(p['cond_b1']),
                    p['cond_W2'], _row(p['cond_b2']))

    lb = linker_batch.astype(jnp.int32).reshape(N, 1)
    nt = linker_node_type.astype(jnp.int32).reshape(N, 1)
    h = _linker_in(x[0], nt, lb, gctx, p['in_W'], _row(p['in_b']))
    for i in range(len(p['conv_W'])):
        a = _agg_call(*h, l_idx)
        h = _conv_tc_ctx(*h, *a, deg_l, lb, gctx,
                         p['conv_W'][i], _row(p['conv_b'][i]),
                         _row(p['ln_g'][i]), _row(p['ln_b'][i]))

    out = _final_tc(*h, p['out_W'], _row(p['out_b']))
    return out[None]


# speedup vs baseline: 5.5182x; 1.0203x over previous
import functools
import math

import jax
import jax.numpy as jnp
from jax import lax
from jax.experimental import pallas as pl
from jax.experimental.pallas import tpu as pltpu
from jax.experimental.pallas import tpu_sc as plsc

N = 50000
E = 800000
G = 128
H = 64
HH = 32
NC, NS, LANES = 2, 16, 16

EPT = E // NS
EPT_PAD = 51200
GEDG = 320
NGRP = EPT_PAD // GEDG
GEDG_D = 1600
NGRP_D = EPT_PAD // GEDG_D
NP = 50048
TROWS = NP // NS
CHUNKS = [(k * GEDG, GEDG) for k in range(9)] + [(9 * GEDG, TROWS - 9 * GEDG)]
SRC_PAD = N
DST_PAD = N

BLK = 2000
GRID = N // BLK

_mesh = plsc.VectorSubcoreMesh(core_axis_name="c", subcore_axis_name="s",
                               num_cores=NC, num_subcores=NS)
_CP = pltpu.CompilerParams(use_tc_tiling_on_sc=False)


def _zero_vmem_rows(buf, nrows, width):
    def body(i, _):
        for w0 in range(0, width, LANES):
            buf[i, pl.ds(w0, LANES)] = jnp.zeros((LANES,), jnp.float32)
        return 0
    lax.fori_loop(0, nrows, body, 0)


def _agg_body(h0, h1, idx_r, o0, o1,
              is0, is1, is2, is3, rows0, rows1, acc,
              isem0, isem1, isem2, isem3, gsem0, gsem1, ssem0, ssem1):
    c = lax.axis_index("c")
    s = lax.axis_index("s")
    islot = (is0, is1, is2, is3)
    isem = (isem0, isem1, isem2, isem3)
    rows = (rows0, rows1)
    gsem = (gsem0, gsem1)
    ssem = (ssem0, ssem1)

    def run(h_ref, out_ref):
        _zero_vmem_rows(rows0, GEDG, HH)
        base = s * TROWS
        for off, ln in CHUNKS:
            pltpu.sync_copy(rows0.at[pl.ds(0, ln)],
                            acc.at[pl.ds(base + off, ln)])
        plsc.subcore_barrier()

        gbase = s * NGRP

        def icopy(g, q):
            pltpu.async_copy(idx_r.at[gbase + g], islot[q], isem[q])

        def idrain(q):
            pltpu.make_async_copy(idx_r.at[gbase], islot[q], isem[q]).wait()

        def gather(q, p):
            pltpu.async_copy(h_ref.at[islot[q].at[0]], rows[p], gsem[p])

        def gwait(p):
            pltpu.make_async_copy(h_ref.at[islot[0].at[0]], rows[p],
                                  gsem[p]).wait()

        def scatter(q, p):
            pltpu.async_copy(rows[p], acc.at[islot[q].at[1]], ssem[p],
                             add=True)

        def sdrain(p):
            pltpu.make_async_copy(rows[p], acc.at[islot[0].at[1]],
                                  ssem[p]).wait()

        def quad(g0, first):
            idrain(0)
            if not first:
                sdrain(0)
            gather(0, 0)
            icopy(g0 + 2, 2)
            idrain(1)
            if not first:
                sdrain(1)
            gather(1, 1)
            icopy(g0 + 3, 3)
            gwait(0)
            scatter(0, 0)
            gwait(1)
            scatter(1, 1)
            idrain(2)
            sdrain(0)
            gather(2, 0)
            icopy(g0 + 4, 0)
            idrain(3)
            sdrain(1)
            gather(3, 1)
            icopy(g0 + 5, 1)
            gwait(0)
            scatter(2, 0)
            gwait(1)
            scatter(3, 1)

        icopy(0, 0)
        icopy(1, 1)
        quad(0, True)

        def step(hb, _):
            quad(4 * hb, False)
            return 0

        lax.fori_loop(1, NGRP // 4, step, 0)
        idrain(0)
        idrain(1)
        sdrain(0)
        sdrain(1)
        plsc.subcore_barrier()

        for off, ln in CHUNKS:
            pltpu.sync_copy(acc.at[pl.ds(base + off, ln)],
                            rows0.at[pl.ds(0, ln)])
            pltpu.sync_copy(rows0.at[pl.ds(0, ln)],
                            out_ref.at[pl.ds(base + off, ln)])

    @pl.when(c == 0)
    def _():
        run(h0, o0)

    @pl.when(c == 1)
    def _():
        run(h1, o1)


_agg_call = pl.kernel(
    _agg_body,
    out_type=(jax.ShapeDtypeStruct((NP, HH), jnp.float32),
              jax.ShapeDtypeStruct((NP, HH), jnp.float32)),
    mesh=_mesh,
    compiler_params=_CP,
    scratch_types=[
        pltpu.VMEM((2, GEDG), jnp.int32),
        pltpu.VMEM((2, GEDG), jnp.int32),
        pltpu.VMEM((2, GEDG), jnp.int32),
        pltpu.VMEM((2, GEDG), jnp.int32),
        pltpu.VMEM((GEDG, HH), jnp.float32),
        pltpu.VMEM((GEDG, HH), jnp.float32),
        pltpu.VMEM_SHARED((NP, HH), jnp.float32),
        pltpu.SemaphoreType.DMA,
        pltpu.SemaphoreType.DMA,
        pltpu.SemaphoreType.DMA,
        pltpu.SemaphoreType.DMA,
        pltpu.SemaphoreType.DMA,
        pltpu.SemaphoreType.DMA,
        pltpu.SemaphoreType.DMA,
        pltpu.SemaphoreType.DMA,
    ],
)


def _deg_body(dst_l, dst_f, out_l, out_f, dv0, dv1, ones_v, zbuf, acc,
              sem0, sem1):
    c = lax.axis_index("c")
    s = lax.axis_index("s")

    def run(dst_r, out_ref):
        io = lax.broadcasted_iota(jnp.int32, (LANES,), 0)
        one_row = jnp.where(io == 0, 1.0, 0.0).astype(jnp.float32)

        def seto(i, _):
            ones_v[i, pl.ds(0, LANES)] = one_row
            return 0
        lax.fori_loop(0, GEDG_D, seto, 0)

        _zero_vmem_rows(zbuf, GEDG, LANES)
        base = s * TROWS
        for off, ln in CHUNKS:
            pltpu.sync_copy(zbuf.at[pl.ds(0, ln)],
                            acc.at[pl.ds(base + off, ln)])
        plsc.subcore_barrier()

        e0 = s * EPT_PAD
        dslot = (dv0, dv1)
        ssem = (sem0, sem1)

        def load(g, q):
            pltpu.sync_copy(dst_r.at[pl.ds(e0 + g * GEDG_D, GEDG_D)],
                            dslot[q])

        def scatter(q):
            pltpu.async_copy(ones_v, acc.at[dslot[q]], ssem[q], add=True)

        def sdrain(q):
            pltpu.make_async_copy(ones_v, acc.at[dslot[q]], ssem[q]).wait()

        load(0, 0)
        scatter(0)
        load(1, 1)
        scatter(1)

        def step(hb, _):
            sdrain(0)
            load(2 * hb, 0)
            scatter(0)
            sdrain(1)
            load(2 * hb + 1, 1)
            scatter(1)
            return 0

        lax.fori_loop(1, NGRP_D // 2, step, 0)
        sdrain(0)
        sdrain(1)
        plsc.subcore_barrier()

        for off, ln in CHUNKS:
            pltpu.sync_copy(acc.at[pl.ds(base + off, ln)],
                            zbuf.at[pl.ds(0, ln)])
            pltpu.sync_copy(zbuf.at[pl.ds(0, ln)],
                            out_ref.at[pl.ds(base + off, ln)])

    @pl.when(c == 0)
    def _():
        run(dst_l, out_l)

    @pl.when(c == 1)
    def _():
        run(dst_f, out_f)


_deg_call = pl.kernel(
    _deg_body,
    out_type=(jax.ShapeDtypeStruct((NP, LANES), jnp.float32),
              jax.ShapeDtypeStruct((NP, LANES), jnp.float32)),
    mesh=_mesh,
    compiler_params=_CP,
    scratch_types=[
        pltpu.VMEM((GEDG_D,), jnp.int32),
        pltpu.VMEM((GEDG_D,), jnp.int32),
        pltpu.VMEM((GEDG_D, LANES), jnp.float32),
        pltpu.VMEM((GEDG, LANES), jnp.float32),
        pltpu.VMEM_SHARED((NP, LANES), jnp.float32),
        pltpu.SemaphoreType.DMA,
        pltpu.SemaphoreType.DMA,
    ],
)


def _prep_edges(src, dst):
    s = src.astype(jnp.int32).reshape(NS, EPT)
    d = dst.astype(jnp.int32).reshape(NS, EPT)
    s = jnp.pad(s, ((0, 0), (0, EPT_PAD - EPT)), constant_values=SRC_PAD)
    d = jnp.pad(d, ((0, 0), (0, EPT_PAD - EPT)), constant_values=DST_PAD)
    packed = jnp.stack([s.reshape(NS, NGRP, GEDG),
                        d.reshape(NS, NGRP, GEDG)], axis=2)
    packed = packed.reshape(NS * NGRP, 2, GEDG)
    packed = jnp.pad(packed, ((0, 2), (0, 0), (0, 0)))
    return packed, d.reshape(NS * EPT_PAD)


def _rowspec(w):
    return pl.BlockSpec((BLK, w), lambda i: (i, 0))


def _fullspec(shape):
    return pl.BlockSpec(shape, lambda i: (0,) * len(shape))


def _ln_relu(y, g, b):
    m = jnp.mean(y, axis=-1, keepdims=True)
    v = jnp.mean((y - m) ** 2, axis=-1, keepdims=True)
    return jnp.maximum((y - m) / jnp.sqrt(v + 1e-5) * g + b, 0.0)


def _conv_tc_body(has_ctx, h0, h1, a0, a1, deg, batch, gctx, W, b, g, bb,
                  o0, o1):
    h = jnp.concatenate([h0[...], h1[...]], axis=1)
    a = jnp.concatenate([a0[...], a1[...]], axis=1)
    d = jnp.maximum(deg[...][:, 0:1], 1.0)
    y = jnp.dot(h + a / d, W[...], preferred_element_type=jnp.float32) + b[...]
    if has_ctx:
        io = lax.broadcasted_iota(jnp.int32, (BLK, G), 1)
        oh = (batch[...] == io).astype(jnp.float32)
        y = y + jnp.dot(oh, gctx[...], preferred_element_type=jnp.float32)
    r = _ln_relu(y, g[...], bb[...])
    o0[...] = r[:, :HH]
    o1[...] = r[:, HH:]


def _make_conv_tc(has_ctx):
    in_specs = [_rowspec(HH)] * 4 + [
        _rowspec(LANES), _rowspec(1), _fullspec((G, H)),
        _fullspec((H, H)), _fullspec((1, H)), _fullspec((1, H)),
        _fullspec((1, H))]
    return pl.pallas_call(
        functools.partial(_conv_tc_body, has_ctx),
        grid=(GRID,),
        in_specs=in_specs,
        out_specs=(_rowspec(HH), _rowspec(HH)),
        out_shape=(jax.ShapeDtypeStruct((NP, HH), jnp.float32),
                   jax.ShapeDtypeStruct((NP, HH), jnp.float32)),
    )


_conv_tc_ctx = _make_conv_tc(True)
_conv_tc_noctx = _make_conv_tc(False)


def _frag_in_body(fx, W, b, o0, o1):
    y = jnp.dot(fx[...], W[...], preferred_element_type=jnp.float32) + b[...]
    o0[...] = y[:, :HH]
    o1[...] = y[:, HH:]


_frag_in = pl.pallas_call(
    _frag_in_body,
    grid=(GRID,),
    in_specs=[_rowspec(4), _fullspec((4, H)), _fullspec((1, H))],
    out_specs=(_rowspec(HH), _rowspec(HH)),
    out_shape=(jax.ShapeDtypeStruct((NP, HH), jnp.float32),
               jax.ShapeDtypeStruct((NP, HH), jnp.float32)),
)


def _linker_in_body(x, nt, batch, gctx, W, b, o0, o1):
    xv = x[...]
    ntv = nt[...]
    y = jnp.dot(xv, W[...][:4, :], preferred_element_type=jnp.float32) + b[...]
    ntc = jnp.clip(ntv, 0, 2)
    for k in range(3):
        y = y + (ntc == k).astype(jnp.float32) * W[...][4 + k:5 + k, :]
    y = y + (ntv > 0).astype(jnp.float32) * W[...][7:8, :]
    io = lax.broadcasted_iota(jnp.int32, (BLK, G), 1)
    oh = (batch[...] == io).astype(jnp.float32)
    y = y + jnp.dot(oh, gctx[...], preferred_element_type=jnp.float32)
    o0[...] = y[:, :HH]
    o1[...] = y[:, HH:]


_linker_in = pl.pallas_call(
    _linker_in_body,
    grid=(GRID,),
    in_specs=[_rowspec(4), _rowspec(1), _rowspec(1), _fullspec((G, H)),
              _fullspec((8, H)), _fullspec((1, H))],
    out_specs=(_rowspec(HH), _rowspec(HH)),
    out_shape=(jax.ShapeDtypeStruct((NP, HH), jnp.float32),
               jax.ShapeDtypeStruct((NP, HH), jnp.float32)),
)


def _frag_tail_body(h0, h1, a0, a1, deg, batch, W, b, g, bb,
                    fW, fb, t, tW1, tb1, tW2, tb2, cW1, cb1, cW2, cb2,
                    gctx_out, sums, cnt):
    i = pl.program_id(0)

    @pl.when(i == 0)
    def _():
        sums[...] = jnp.zeros_like(sums)
        cnt[...] = jnp.zeros_like(cnt)

    h = jnp.concatenate([h0[...], h1[...]], axis=1)
    a = jnp.concatenate([a0[...], a1[...]], axis=1)
    d = jnp.maximum(deg[...][:, 0:1], 1.0)
    y = jnp.dot(h + a / d, W[...], preferred_element_type=jnp.float32) + b[...]
    r = _ln_relu(y, g[...], bb[...])
    io = lax.broadcasted_iota(jnp.int32, (BLK, 2 * G), 1)
    oh = (batch[...] == io).astype(jnp.float32)
    dn = (((0,), (0,)), ((), ()))
    sums[...] += lax.dot_general(oh, r, dn,
                                 preferred_element_type=jnp.float32)
    cnt[...] += lax.dot_general(oh, jnp.ones((BLK, 8), jnp.float32), dn,
                                preferred_element_type=jnp.float32)

    @pl.when(i == GRID - 1)
    def _():
        pooled = sums[...] / jnp.maximum(cnt[...][:, 0:1], 1.0)
        fctx = jnp.dot(pooled, fW[...],
                       preferred_element_type=jnp.float32) + fb[...]
        left = fctx[:G, :]
        right = fctx[G:, :]
        j = lax.broadcasted_iota(jnp.int32, (1, H), 1).astype(jnp.float32)
        freqs = jnp.exp(-math.log(10000.0) * j / 64.0)
        ang = t[...] * freqs
        te = jnp.concatenate([jnp.sin(ang), jnp.cos(ang)], axis=1)
        z = jnp.dot(te, tW1[...], preferred_element_type=jnp.float32) + tb1[...]
        z = z * lax.logistic(z)
        th = jnp.dot(z, tW2[...], preferred_element_type=jnp.float32) + tb2[...]
        ci = jnp.concatenate([left, right, th], axis=1)
        z2 = jnp.dot(ci, cW1[...], preferred_element_type=jnp.float32) + cb1[...]
        z2 = z2 * lax.logistic(z2)
        gctx_out[...] = jnp.dot(z2, cW2[...],
                                preferred_element_type=jnp.float32) + cb2[...]


_frag_tail_tc = pl.pallas_call(
    _frag_tail_body,
    grid=(GRID,),
    in_specs=[_rowspec(HH), _rowspec(HH), _rowspec(HH), _rowspec(HH),
              _rowspec(LANES), _rowspec(1),
              _fullspec((H, H)), _fullspec((1, H)), _fullspec((1, H)),
              _fullspec((1, H)),
              _fullspec((H, H)), _fullspec((1, H)), _fullspec((G, 1)),
              _fullspec((2 * H, H)), _fullspec((1, H)), _fullspec((H, H)),
              _fullspec((1, H)), _fullspec((3 * H, H)), _fullspec((1, H)),
              _fullspec((H, H)), _fullspec((1, H))],
    out_specs=_fullspec((G, H)),
    out_shape=jax.ShapeDtypeStruct((G, H), jnp.float32),
    scratch_shapes=[pltpu.VMEM((2 * G, H), jnp.float32),
                    pltpu.VMEM((2 * G, 8), jnp.float32)],
)


def _linker_tail_body(h0, h1, a0, a1, deg, batch, gctx, W, b, g, bb, oW, ob,
                      out):
    h = jnp.concatenate([h0[...], h1[...]], axis=1)
    a = jnp.concatenate([a0[...], a1[...]], axis=1)
    d = jnp.maximum(deg[...][:, 0:1], 1.0)
    y = jnp.dot(h + a / d, W[...], preferred_element_type=jnp.float32) + b[...]
    io = lax.broadcasted_iota(jnp.int32, (BLK, G), 1)
    oh = (batch[...] == io).astype(jnp.float32)
    y = y + jnp.dot(oh, gctx[...], preferred_element_type=jnp.float32)
    r = _ln_relu(y, g[...], bb[...])
    out[...] = jnp.dot(r, oW[...], preferred_element_type=jnp.float32) + ob[...]


_linker_tail_tc = pl.pallas_call(
    _linker_tail_body,
    grid=(GRID,),
    in_specs=[_rowspec(HH), _rowspec(HH), _rowspec(HH), _rowspec(HH),
              _rowspec(LANES), _rowspec(1), _fullspec((G, H)),
              _fullspec((H, H)), _fullspec((1, H)), _fullspec((1, H)),
              _fullspec((1, H)), _fullspec((H, 4)), _fullspec((1, 4))],
    out_specs=_rowspec(4),
    out_shape=jax.ShapeDtypeStruct((N, 4), jnp.float32),
)


def _row(v):
    return v.reshape(1, -1).astype(jnp.float32)


def kernel(x, t, linker_batch, linker_graph_ptr, linker_node_type,
           linker_edge_index, left_x, left_edge_index, left_batch, right_x,
           right_edge_index, right_batch, params):
    NF = left_x.shape[0]
    p = params
    fp = p['frag']

    fx = jnp.concatenate([left_x, right_x], axis=0)
    f_src = jnp.concatenate([left_edge_index[0], right_edge_index[0] + NF])
    f_dst = jnp.concatenate([left_edge_index[1], right_edge_index[1] + NF])
    f_batch = jnp.concatenate([left_batch, right_batch + G])
    f_idx, f_dflat = _prep_edges(f_src, f_dst)
    l_idx, l_dflat = _prep_edges(linker_edge_index[0], linker_edge_index[1])

    deg_l, deg_f = _deg_call(l_dflat, f_dflat)

    zero_batch = jnp.zeros((N, 1), jnp.int32)
    zero_gctx = jnp.zeros((G, H), jnp.float32)

    f_batch2 = f_batch.astype(jnp.int32).reshape(N, 1)
    nfc = len(fp['conv_W'])
    fh = _frag_in(fx, fp['in_W'], _row(fp['in_b']))
    for i in range(nfc - 1):
        fa = _agg_call(*fh, f_idx)
        fh = _conv_tc_noctx(*fh, *fa, deg_f, zero_batch, zero_gctx,
                            fp['conv_W'][i], _row(fp['conv_b'][i]),
                            _row(fp['ln_g'][i]), _row(fp['ln_b'][i]))
    fa = _agg_call(*fh, f_idx)
    gctx = _frag_tail_tc(*fh, *fa, deg_f, f_batch2,
                         fp['conv_W'][nfc - 1], _row(fp['conv_b'][nfc - 1]),
                         _row(fp['ln_g'][nfc - 1]), _row(fp['ln_b'][nfc - 1]),
                         fp['out_W'], _row(fp['out_b']),
                         t.reshape(G, 1).astype(jnp.float32),
                         p['time_W1'], _row(p['time_b1']),
                         p['time_W2'], _row(p['time_b2']),
                         p['cond_W1'], _row(p['cond_b1']),
                         p['cond_W2'], _row(p['cond_b2']))

    lb = linker_batch.astype(jnp.int32).reshape(N, 1)
    nt = linker_node_type.astype(jnp.int32).reshape(N, 1)
    nlc = len(p['conv_W'])
    h = _linker_in(x[0], nt, lb, gctx, p['in_W'], _row(p['in_b']))
    for i in range(nlc - 1):
        a = _agg_call(*h, l_idx)
        h = _conv_tc_ctx(*h, *a, deg_l, lb, gctx,
                         p['conv_W'][i], _row(p['conv_b'][i]),
                         _row(p['ln_g'][i]), _row(p['ln_b'][i]))
    a = _agg_call(*h, l_idx)
    out = _linker_tail_tc(*h, *a, deg_l, lb, gctx,
                          p['conv_W'][nlc - 1], _row(p['conv_b'][nlc - 1]),
                          _row(p['ln_g'][nlc - 1]), _row(p['ln_b'][nlc - 1]),
                          p['out_W'], _row(p['out_b']))
    return out[None]


# speedup vs baseline: 5.6808x; 1.0295x over previous
import functools
import math

import jax
import jax.numpy as jnp
from jax import lax
from jax.experimental import pallas as pl
from jax.experimental.pallas import tpu as pltpu
from jax.experimental.pallas import tpu_sc as plsc

N = 50000
E = 800000
G = 128
H = 64
HH = 32
NC, NS, LANES = 2, 16, 16

EPT = E // NS
EPT_PAD = 51200
GEDG = 160
NGRP = EPT_PAD // GEDG
NRB = 4
NIS = 8
GEDG_D = 1600
NGRP_D = EPT_PAD // GEDG_D
NP = 50048
TROWS = NP // NS
SRC_PAD = N
DST_PAD = N

BLK = 2000
GRID = N // BLK

_mesh = plsc.VectorSubcoreMesh(core_axis_name="c", subcore_axis_name="s",
                               num_cores=NC, num_subcores=NS)
_CP = pltpu.CompilerParams(use_tc_tiling_on_sc=False)


def _zero_vmem_rows(buf, nrows, width):
    def body(i, _):
        for w0 in range(0, width, LANES):
            buf[i, pl.ds(w0, LANES)] = jnp.zeros((LANES,), jnp.float32)
        return 0
    lax.fori_loop(0, nrows, body, 0)


ZCHUNKS = [(k * GEDG, GEDG) for k in range(19)] + [(19 * GEDG, TROWS - 19 * GEDG)]


def _agg_body(h0, h1, idx_r, o0, o1,
              is0, is1, is2, is3, is4, is5, is6, is7,
              rw0, rw1, rw2, rw3, acc,
              im0, im1, im2, im3, im4, im5, im6, im7,
              gm0, gm1, gm2, gm3, sm0, sm1, sm2, sm3):
    c = lax.axis_index("c")
    s = lax.axis_index("s")
    islot = (is0, is1, is2, is3, is4, is5, is6, is7)
    isem = (im0, im1, im2, im3, im4, im5, im6, im7)
    rows = (rw0, rw1, rw2, rw3)
    gsem = (gm0, gm1, gm2, gm3)
    ssem = (sm0, sm1, sm2, sm3)

    def run(h_ref, out_ref):
        base = s * TROWS
        for p in range(NRB):
            _zero_vmem_rows(rows[p], GEDG, HH)
        for k, (off, ln) in enumerate(ZCHUNKS):
            pltpu.async_copy(rows[k % NRB].at[pl.ds(0, ln)],
                             acc.at[pl.ds(base + off, ln)], gm0)
        for off, ln in ZCHUNKS:
            pltpu.make_async_copy(rows[0].at[pl.ds(0, ln)],
                                  acc.at[pl.ds(base, ln)], gm0).wait()
        plsc.subcore_barrier()

        gbase = s * NGRP

        def icopy(g, q):
            pltpu.async_copy(idx_r.at[gbase + g], islot[q], isem[q])

        def idrain(q):
            pltpu.make_async_copy(idx_r.at[gbase], islot[q], isem[q]).wait()

        def gather(q, p):
            pltpu.async_copy(h_ref.at[islot[q].at[0]], rows[p], gsem[p])

        def gwait(p):
            pltpu.make_async_copy(h_ref.at[islot[0].at[0]], rows[p],
                                  gsem[p]).wait()

        def scatter(q, p):
            pltpu.async_copy(rows[p], acc.at[islot[q].at[1]], ssem[p],
                             add=True)

        def sdrain(p):
            pltpu.make_async_copy(rows[0], acc.at[islot[0].at[1]],
                                  ssem[p]).wait()

        def step(gexpr, j, do_sdrain=True):
            idrain((j + 1) % NIS)
            if do_sdrain:
                sdrain((j + 1) % NRB)
            gather((j + 1) % NIS, (j + 1) % NRB)
            icopy(gexpr + 5, (j + 5) % NIS)
            gwait(j % NRB)
            scatter(j % NIS, j % NRB)

        for g in range(5):
            icopy(g, g)
        idrain(0)
        gather(0, 0)
        for j in range(8):
            step(j, j, do_sdrain=(j >= 3))

        def octv(ob, _):
            g0 = 8 * ob
            for j in range(8):
                step(g0 + j, j)
            return 0

        lax.fori_loop(1, NGRP // 8, octv, 0)
        gwait(0)
        sdrain(1)
        sdrain(2)
        sdrain(3)
        idrain(1)
        idrain(2)
        idrain(3)
        idrain(4)
        plsc.subcore_barrier()

        nz = len(ZCHUNKS)
        for k, (off, ln) in enumerate(ZCHUNKS):
            p = k % NRB
            if k >= NRB:
                pln = ZCHUNKS[k - NRB][1]
                pltpu.make_async_copy(rows[p].at[pl.ds(0, pln)],
                                      out_ref.at[pl.ds(base, pln)],
                                      ssem[p]).wait()
            pltpu.sync_copy(acc.at[pl.ds(base + off, ln)],
                            rows[p].at[pl.ds(0, ln)])
            pltpu.async_copy(rows[p].at[pl.ds(0, ln)],
                             out_ref.at[pl.ds(base + off, ln)], ssem[p])
        for k in range(nz - NRB, nz):
            p = k % NRB
            ln = ZCHUNKS[k][1]
            pltpu.make_async_copy(rows[p].at[pl.ds(0, ln)],
                                  out_ref.at[pl.ds(base, ln)],
                                  ssem[p]).wait()

    @pl.when(c == 0)
    def _():
        run(h0, o0)

    @pl.when(c == 1)
    def _():
        run(h1, o1)


_agg_call = pl.kernel(
    _agg_body,
    out_type=(jax.ShapeDtypeStruct((NP, HH), jnp.float32),
              jax.ShapeDtypeStruct((NP, HH), jnp.float32)),
    mesh=_mesh,
    compiler_params=_CP,
    scratch_types=(
        [pltpu.VMEM((2, GEDG), jnp.int32) for _ in range(NIS)]
        + [pltpu.VMEM((GEDG, HH), jnp.float32) for _ in range(NRB)]
        + [pltpu.VMEM_SHARED((NP, HH), jnp.float32)]
        + [pltpu.SemaphoreType.DMA] * (NIS + 2 * NRB)
    ),
)


def _deg_body(dst_l, dst_f, out_l, out_f, dv0, dv1, ones_v, zbuf, acc,
              sem0, sem1):
    c = lax.axis_index("c")
    s = lax.axis_index("s")

    def run(dst_r, out_ref):
        io = lax.broadcasted_iota(jnp.int32, (LANES,), 0)
        one_row = jnp.where(io == 0, 1.0, 0.0).astype(jnp.float32)

        def seto(i, _):
            ones_v[i, pl.ds(0, LANES)] = one_row
            return 0
        lax.fori_loop(0, GEDG_D, seto, 0)

        _zero_vmem_rows(zbuf, GEDG, LANES)
        base = s * TROWS
        for off, ln in ZCHUNKS:
            pltpu.sync_copy(zbuf.at[pl.ds(0, ln)],
                            acc.at[pl.ds(base + off, ln)])
        plsc.subcore_barrier()

        e0 = s * EPT_PAD
        dslot = (dv0, dv1)
        ssem = (sem0, sem1)

        def load(g, q):
            pltpu.sync_copy(dst_r.at[pl.ds(e0 + g * GEDG_D, GEDG_D)],
                            dslot[q])

        def scatter(q):
            pltpu.async_copy(ones_v, acc.at[dslot[q]], ssem[q], add=True)

        def sdrain(q):
            pltpu.make_async_copy(ones_v, acc.at[dslot[q]], ssem[q]).wait()

        load(0, 0)
        scatter(0)
        load(1, 1)
        scatter(1)

        def step(hb, _):
            sdrain(0)
            load(2 * hb, 0)
            scatter(0)
            sdrain(1)
            load(2 * hb + 1, 1)
            scatter(1)
            return 0

        lax.fori_loop(1, NGRP_D // 2, step, 0)
        sdrain(0)
        sdrain(1)
        plsc.subcore_barrier()

        for off, ln in ZCHUNKS:
            pltpu.sync_copy(acc.at[pl.ds(base + off, ln)],
                            zbuf.at[pl.ds(0, ln)])
            pltpu.sync_copy(zbuf.at[pl.ds(0, ln)],
                            out_ref.at[pl.ds(base + off, ln)])

    @pl.when(c == 0)
    def _():
        run(dst_l, out_l)

    @pl.when(c == 1)
    def _():
        run(dst_f, out_f)


_deg_call = pl.kernel(
    _deg_body,
    out_type=(jax.ShapeDtypeStruct((NP, LANES), jnp.float32),
              jax.ShapeDtypeStruct((NP, LANES), jnp.float32)),
    mesh=_mesh,
    compiler_params=_CP,
    scratch_types=[
        pltpu.VMEM((GEDG_D,), jnp.int32),
        pltpu.VMEM((GEDG_D,), jnp.int32),
        pltpu.VMEM((GEDG_D, LANES), jnp.float32),
        pltpu.VMEM((GEDG, LANES), jnp.float32),
        pltpu.VMEM_SHARED((NP, LANES), jnp.float32),
        pltpu.SemaphoreType.DMA,
        pltpu.SemaphoreType.DMA,
    ],
)


def _prep_edges(src, dst):
    s = src.astype(jnp.int32).reshape(NS, EPT)
    d = dst.astype(jnp.int32).reshape(NS, EPT)
    s = jnp.pad(s, ((0, 0), (0, EPT_PAD - EPT)), constant_values=SRC_PAD)
    d = jnp.pad(d, ((0, 0), (0, EPT_PAD - EPT)), constant_values=DST_PAD)
    packed = jnp.stack([s.reshape(NS, NGRP, GEDG),
                        d.reshape(NS, NGRP, GEDG)], axis=2)
    packed = packed.reshape(NS * NGRP, 2, GEDG)
    packed = jnp.pad(packed, ((0, 8), (0, 0), (0, 0)))
    return packed, d.reshape(NS * EPT_PAD)


def _rowspec(w):
    return pl.BlockSpec((BLK, w), lambda i: (i, 0))


def _fullspec(shape):
    return pl.BlockSpec(shape, lambda i: (0,) * len(shape))


def _ln_relu(y, g, b):
    m = jnp.mean(y, axis=-1, keepdims=True)
    v = jnp.mean((y - m) ** 2, axis=-1, keepdims=True)
    return jnp.maximum((y - m) / jnp.sqrt(v + 1e-5) * g + b, 0.0)


def _conv_tc_body(has_ctx, h0, h1, a0, a1, deg, batch, gctx, W, b, g, bb,
                  o0, o1):
    h = jnp.concatenate([h0[...], h1[...]], axis=1)
    a = jnp.concatenate([a0[...], a1[...]], axis=1)
    d = jnp.maximum(deg[...][:, 0:1], 1.0)
    y = jnp.dot(h + a / d, W[...], preferred_element_type=jnp.float32) + b[...]
    if has_ctx:
        io = lax.broadcasted_iota(jnp.int32, (BLK, G), 1)
        oh = (batch[...] == io).astype(jnp.float32)
        y = y + jnp.dot(oh, gctx[...], preferred_element_type=jnp.float32)
    r = _ln_relu(y, g[...], bb[...])
    o0[...] = r[:, :HH]
    o1[...] = r[:, HH:]


def _make_conv_tc(has_ctx):
    in_specs = [_rowspec(HH)] * 4 + [
        _rowspec(LANES), _rowspec(1), _fullspec((G, H)),
        _fullspec((H, H)), _fullspec((1, H)), _fullspec((1, H)),
        _fullspec((1, H))]
    return pl.pallas_call(
        functools.partial(_conv_tc_body, has_ctx),
        grid=(GRID,),
        in_specs=in_specs,
        out_specs=(_rowspec(HH), _rowspec(HH)),
        out_shape=(jax.ShapeDtypeStruct((NP, HH), jnp.float32),
                   jax.ShapeDtypeStruct((NP, HH), jnp.float32)),
    )


_conv_tc_ctx = _make_conv_tc(True)
_conv_tc_noctx = _make_conv_tc(False)


def _frag_in_body(fx, W, b, o0, o1):
    y = jnp.dot(fx[...], W[...], preferred_element_type=jnp.float32) + b[...]
    o0[...] = y[:, :HH]
    o1[...] = y[:, HH:]


_frag_in = pl.pallas_call(
    _frag_in_body,
    grid=(GRID,),
    in_specs=[_rowspec(4), _fullspec((4, H)), _fullspec((1, H))],
    out_specs=(_rowspec(HH), _rowspec(HH)),
    out_shape=(jax.ShapeDtypeStruct((NP, HH), jnp.float32),
               jax.ShapeDtypeStruct((NP, HH), jnp.float32)),
)


def _linker_in_body(x, nt, batch, gctx, W, b, o0, o1):
    xv = x[...]
    ntv = nt[...]
    y = jnp.dot(xv, W[...][:4, :], preferred_element_type=jnp.float32) + b[...]
    ntc = jnp.clip(ntv, 0, 2)
    for k in range(3):
        y = y + (ntc == k).astype(jnp.float32) * W[...][4 + k:5 + k, :]
    y = y + (ntv > 0).astype(jnp.float32) * W[...][7:8, :]
    io = lax.broadcasted_iota(jnp.int32, (BLK, G), 1)
    oh = (batch[...] == io).astype(jnp.float32)
    y = y + jnp.dot(oh, gctx[...], preferred_element_type=jnp.float32)
    o0[...] = y[:, :HH]
    o1[...] = y[:, HH:]


_linker_in = pl.pallas_call(
    _linker_in_body,
    grid=(GRID,),
    in_specs=[_rowspec(4), _rowspec(1), _rowspec(1), _fullspec((G, H)),
              _fullspec((8, H)), _fullspec((1, H))],
    out_specs=(_rowspec(HH), _rowspec(HH)),
    out_shape=(jax.ShapeDtypeStruct((NP, HH), jnp.float32),
               jax.ShapeDtypeStruct((NP, HH), jnp.float32)),
)


def _frag_tail_body(h0, h1, a0, a1, deg, batch, W, b, g, bb,
                    fW, fb, t, tW1, tb1, tW2, tb2, cW1, cb1, cW2, cb2,
                    gctx_out, sums, cnt):
    i = pl.program_id(0)

    @pl.when(i == 0)
    def _():
        sums[...] = jnp.zeros_like(sums)
        cnt[...] = jnp.zeros_like(cnt)

    h = jnp.concatenate([h0[...], h1[...]], axis=1)
    a = jnp.concatenate([a0[...], a1[...]], axis=1)
    d = jnp.maximum(deg[...][:, 0:1], 1.0)
    y = jnp.dot(h + a / d, W[...], preferred_element_type=jnp.float32) + b[...]
    r = _ln_relu(y, g[...], bb[...])
    io = lax.broadcasted_iota(jnp.int32, (BLK, 2 * G), 1)
    oh = (batch[...] == io).astype(jnp.float32)
    dn = (((0,), (0,)), ((), ()))
    sums[...] += lax.dot_general(oh, r, dn,
                                 preferred_element_type=jnp.float32)
    cnt[...] += lax.dot_general(oh, jnp.ones((BLK, 8), jnp.float32), dn,
                                preferred_element_type=jnp.float32)

    @pl.when(i == GRID - 1)
    def _():
        pooled = sums[...] / jnp.maximum(cnt[...][:, 0:1], 1.0)
        fctx = jnp.dot(pooled, fW[...],
                       preferred_element_type=jnp.float32) + fb[...]
        left = fctx[:G, :]
        right = fctx[G:, :]
        j = lax.broadcasted_iota(jnp.int32, (1, H), 1).astype(jnp.float32)
        freqs = jnp.exp(-math.log(10000.0) * j / 64.0)
        ang = t[...] * freqs
        te = jnp.concatenate([jnp.sin(ang), jnp.cos(ang)], axis=1)
        z = jnp.dot(te, tW1[...], preferred_element_type=jnp.float32) + tb1[...]
        z = z * lax.logistic(z)
        th = jnp.dot(z, tW2[...], preferred_element_type=jnp.float32) + tb2[...]
        ci = jnp.concatenate([left, right, th], axis=1)
        z2 = jnp.dot(ci, cW1[...], preferred_element_type=jnp.float32) + cb1[...]
        z2 = z2 * lax.logistic(z2)
        gctx_out[...] = jnp.dot(z2, cW2[...],
                                preferred_element_type=jnp.float32) + cb2[...]


_frag_tail_tc = pl.pallas_call(
    _frag_tail_body,
    grid=(GRID,),
    in_specs=[_rowspec(HH), _rowspec(HH), _rowspec(HH), _rowspec(HH),
              _rowspec(LANES), _rowspec(1),
              _fullspec((H, H)), _fullspec((1, H)), _fullspec((1, H)),
              _fullspec((1, H)),
              _fullspec((H, H)), _fullspec((1, H)), _fullspec((G, 1)),
              _fullspec((2 * H, H)), _fullspec((1, H)), _fullspec((H, H)),
              _fullspec((1, H)), _fullspec((3 * H, H)), _fullspec((1, H)),
              _fullspec((H, H)), _fullspec((1, H))],
    out_specs=_fullspec((G, H)),
    out_shape=jax.ShapeDtypeStruct((G, H), jnp.float32),
    scratch_shapes=[pltpu.VMEM((2 * G, H), jnp.float32),
                    pltpu.VMEM((2 * G, 8), jnp.float32)],
)


def _linker_tail_body(h0, h1, a0, a1, deg, batch, gctx, W, b, g, bb, oW, ob,
                      out):
    h = jnp.concatenate([h0[...], h1[...]], axis=1)
    a = jnp.concatenate([a0[...], a1[...]], axis=1)
    d = jnp.maximum(deg[...][:, 0:1], 1.0)
    y = jnp.dot(h + a / d, W[...], preferred_element_type=jnp.float32) + b[...]
    io = lax.broadcasted_iota(jnp.int32, (BLK, G), 1)
    oh = (batch[...] == io).astype(jnp.float32)
    y = y + jnp.dot(oh, gctx[...], preferred_element_type=jnp.float32)
    r = _ln_relu(y, g[...], bb[...])
    out[...] = jnp.dot(r, oW[...], preferred_element_type=jnp.float32) + ob[...]


_linker_tail_tc = pl.pallas_call(
    _linker_tail_body,
    grid=(GRID,),
    in_specs=[_rowspec(HH), _rowspec(HH), _rowspec(HH), _rowspec(HH),
              _rowspec(LANES), _rowspec(1), _fullspec((G, H)),
              _fullspec((H, H)), _fullspec((1, H)), _fullspec((1, H)),
              _fullspec((1, H)), _fullspec((H, 4)), _fullspec((1, 4))],
    out_specs=_rowspec(4),
    out_shape=jax.ShapeDtypeStruct((N, 4), jnp.float32),
)


def _row(v):
    return v.reshape(1, -1).astype(jnp.float32)


def kernel(x, t, linker_batch, linker_graph_ptr, linker_node_type,
           linker_edge_index, left_x, left_edge_index, left_batch, right_x,
           right_edge_index, right_batch, params):
    NF = left_x.shape[0]
    p = params
    fp = p['frag']

    fx = jnp.concatenate([left_x, right_x], axis=0)
    f_src = jnp.concatenate([left_edge_index[0], right_edge_index[0] + NF])
    f_dst = jnp.concatenate([left_edge_index[1], right_edge_index[1] + NF])
    f_batch = jnp.concatenate([left_batch, right_batch + G])
    f_idx, f_dflat = _prep_edges(f_src, f_dst)
    l_idx, l_dflat = _prep_edges(linker_edge_index[0], linker_edge_index[1])

    deg_l, deg_f = _deg_call(l_dflat, f_dflat)

    zero_batch = jnp.zeros((N, 1), jnp.int32)
    zero_gctx = jnp.zeros((G, H), jnp.float32)

    f_batch2 = f_batch.astype(jnp.int32).reshape(N, 1)
    nfc = len(fp['conv_W'])
    fh = _frag_in(fx, fp['in_W'], _row(fp['in_b']))
    for i in range(nfc - 1):
        fa = _agg_call(*fh, f_idx)
        fh = _conv_tc_noctx(*fh, *fa, deg_f, zero_batch, zero_gctx,
                            fp['conv_W'][i], _row(fp['conv_b'][i]),
                            _row(fp['ln_g'][i]), _row(fp['ln_b'][i]))
    fa = _agg_call(*fh, f_idx)
    gctx = _frag_tail_tc(*fh, *fa, deg_f, f_batch2,
                         fp['conv_W'][nfc - 1], _row(fp['conv_b'][nfc - 1]),
                         _row(fp['ln_g'][nfc - 1]), _row(fp['ln_b'][nfc - 1]),
                         fp['out_W'], _row(fp['out_b']),
                         t.reshape(G, 1).astype(jnp.float32),
                         p['time_W1'], _row(p['time_b1']),
                         p['time_W2'], _row(p['time_b2']),
                         p['cond_W1'], _row(p['cond_b1']),
                         p['cond_W2'], _row(p['cond_b2']))

    lb = linker_batch.astype(jnp.int32).reshape(N, 1)
    nt = linker_node_type.astype(jnp.int32).reshape(N, 1)
    nlc = len(p['conv_W'])
    h = _linker_in(x[0], nt, lb, gctx, p['in_W'], _row(p['in_b']))
    for i in range(nlc - 1):
        a = _agg_call(*h, l_idx)
        h = _conv_tc_ctx(*h, *a, deg_l, lb, gctx,
                         p['conv_W'][i], _row(p['conv_b'][i]),
                         _row(p['ln_g'][i]), _row(p['ln_b'][i]))
    a = _agg_call(*h, l_idx)
    out = _linker_tail_tc(*h, *a, deg_l, lb, gctx,
                          p['conv_W'][nlc - 1], _row(p['conv_b'][nlc - 1]),
                          _row(p['ln_g'][nlc - 1]), _row(p['ln_b'][nlc - 1]),
                          p['out_W'], _row(p['out_b']))
    return out[None]


# speedup vs baseline: 5.7351x; 1.0096x over previous
import functools
import math

import jax
import jax.numpy as jnp
from jax import lax
from jax.experimental import pallas as pl
from jax.experimental.pallas import tpu as pltpu
from jax.experimental.pallas import tpu_sc as plsc

N = 50000
E = 800000
G = 128
H = 64
HH = 32
NC, NS, LANES = 2, 16, 16

EPT = E // NS
EPT_PAD = 51200
GEDG = 160
NGRP = EPT_PAD // GEDG
NRB = 4
NIS = 8
GEDG_D = 1600
NGRP_D = EPT_PAD // GEDG_D
NP = 50048
TROWS = NP // NS
SRC_PAD = N
DST_PAD = N

BLK = 5000
GRID = N // BLK

_mesh = plsc.VectorSubcoreMesh(core_axis_name="c", subcore_axis_name="s",
                               num_cores=NC, num_subcores=NS)
_CP = pltpu.CompilerParams(use_tc_tiling_on_sc=False)


def _zero_vmem_rows(buf, nrows, width):
    def body(i, _):
        for w0 in range(0, width, LANES):
            buf[i, pl.ds(w0, LANES)] = jnp.zeros((LANES,), jnp.float32)
        return 0
    lax.fori_loop(0, nrows, body, 0)


ZCHUNKS = [(k * GEDG, GEDG) for k in range(19)] + [(19 * GEDG, TROWS - 19 * GEDG)]


def _agg_body(h0, h1, idx_r, o0, o1,
              is0, is1, is2, is3, is4, is5, is6, is7,
              rw0, rw1, rw2, rw3, acc,
              im0, im1, im2, im3, im4, im5, im6, im7,
              gm0, gm1, gm2, gm3, sm0, sm1, sm2, sm3):
    c = lax.axis_index("c")
    s = lax.axis_index("s")
    islot = (is0, is1, is2, is3, is4, is5, is6, is7)
    isem = (im0, im1, im2, im3, im4, im5, im6, im7)
    rows = (rw0, rw1, rw2, rw3)
    gsem = (gm0, gm1, gm2, gm3)
    ssem = (sm0, sm1, sm2, sm3)

    def run(h_ref, out_ref):
        base = s * TROWS
        for p in range(NRB):
            _zero_vmem_rows(rows[p], GEDG, HH)
        for k, (off, ln) in enumerate(ZCHUNKS):
            pltpu.async_copy(rows[k % NRB].at[pl.ds(0, ln)],
                             acc.at[pl.ds(base + off, ln)], gm0)
        for off, ln in ZCHUNKS:
            pltpu.make_async_copy(rows[0].at[pl.ds(0, ln)],
                                  acc.at[pl.ds(base, ln)], gm0).wait()
        plsc.subcore_barrier()

        gbase = s * NGRP

        def icopy(g, q):
            pltpu.async_copy(idx_r.at[gbase + g], islot[q], isem[q])

        def idrain(q):
            pltpu.make_async_copy(idx_r.at[gbase], islot[q], isem[q]).wait()

        def gather(q, p):
            pltpu.async_copy(h_ref.at[islot[q].at[0]], rows[p], gsem[p])

        def gwait(p):
            pltpu.make_async_copy(h_ref.at[islot[0].at[0]], rows[p],
                                  gsem[p]).wait()

        def scatter(q, p):
            pltpu.async_copy(rows[p], acc.at[islot[q].at[1]], ssem[p],
                             add=True)

        def sdrain(p):
            pltpu.make_async_copy(rows[0], acc.at[islot[0].at[1]],
                                  ssem[p]).wait()

        def step(gexpr, j, do_sdrain=True):
            idrain((j + 1) % NIS)
            if do_sdrain:
                sdrain((j + 1) % NRB)
            gather((j + 1) % NIS, (j + 1) % NRB)
            icopy(gexpr + 5, (j + 5) % NIS)
            gwait(j % NRB)
            scatter(j % NIS, j % NRB)

        for g in range(5):
            icopy(g, g)
        idrain(0)
        gather(0, 0)
        for j in range(8):
            step(j, j, do_sdrain=(j >= 3))

        def octv(ob, _):
            g0 = 8 * ob
            for j in range(8):
                step(g0 + j, j)
            return 0

        lax.fori_loop(1, NGRP // 8, octv, 0)
        gwait(0)
        sdrain(1)
        sdrain(2)
        sdrain(3)
        idrain(1)
        idrain(2)
        idrain(3)
        idrain(4)
        plsc.subcore_barrier()

        nz = len(ZCHUNKS)
        for k, (off, ln) in enumerate(ZCHUNKS):
            p = k % NRB
            if k >= NRB:
                pln = ZCHUNKS[k - NRB][1]
                pltpu.make_async_copy(rows[p].at[pl.ds(0, pln)],
                                      out_ref.at[pl.ds(base, pln)],
                                      ssem[p]).wait()
            pltpu.sync_copy(acc.at[pl.ds(base + off, ln)],
                            rows[p].at[pl.ds(0, ln)])
            pltpu.async_copy(rows[p].at[pl.ds(0, ln)],
                             out_ref.at[pl.ds(base + off, ln)], ssem[p])
        for k in range(nz - NRB, nz):
            p = k % NRB
            ln = ZCHUNKS[k][1]
            pltpu.make_async_copy(rows[p].at[pl.ds(0, ln)],
                                  out_ref.at[pl.ds(base, ln)],
                                  ssem[p]).wait()

    @pl.when(c == 0)
    def _():
        run(h0, o0)

    @pl.when(c == 1)
    def _():
        run(h1, o1)


_agg_call = pl.kernel(
    _agg_body,
    out_type=(jax.ShapeDtypeStruct((NP, HH), jnp.float32),
              jax.ShapeDtypeStruct((NP, HH), jnp.float32)),
    mesh=_mesh,
    compiler_params=_CP,
    scratch_types=(
        [pltpu.VMEM((2, GEDG), jnp.int32) for _ in range(NIS)]
        + [pltpu.VMEM((GEDG, HH), jnp.float32) for _ in range(NRB)]
        + [pltpu.VMEM_SHARED((NP, HH), jnp.float32)]
        + [pltpu.SemaphoreType.DMA] * (NIS + 2 * NRB)
    ),
)


def _deg_body(dst_l, dst_f, out_l, out_f, dv0, dv1, ones_v, zbuf, acc,
              sem0, sem1):
    c = lax.axis_index("c")
    s = lax.axis_index("s")

    def run(dst_r, out_ref):
        io = lax.broadcasted_iota(jnp.int32, (LANES,), 0)
        one_row = jnp.where(io == 0, 1.0, 0.0).astype(jnp.float32)

        def seto(i, _):
            ones_v[i, pl.ds(0, LANES)] = one_row
            return 0
        lax.fori_loop(0, GEDG_D, seto, 0)

        _zero_vmem_rows(zbuf, GEDG, LANES)
        base = s * TROWS
        for off, ln in ZCHUNKS:
            pltpu.sync_copy(zbuf.at[pl.ds(0, ln)],
                            acc.at[pl.ds(base + off, ln)])
        plsc.subcore_barrier()

        e0 = s * EPT_PAD
        dslot = (dv0, dv1)
        ssem = (sem0, sem1)

        def load(g, q):
            pltpu.sync_copy(dst_r.at[pl.ds(e0 + g * GEDG_D, GEDG_D)],
                            dslot[q])

        def scatter(q):
            pltpu.async_copy(ones_v, acc.at[dslot[q]], ssem[q], add=True)

        def sdrain(q):
            pltpu.make_async_copy(ones_v, acc.at[dslot[q]], ssem[q]).wait()

        load(0, 0)
        scatter(0)
        load(1, 1)
        scatter(1)

        def step(hb, _):
            sdrain(0)
            load(2 * hb, 0)
            scatter(0)
            sdrain(1)
            load(2 * hb + 1, 1)
            scatter(1)
            return 0

        lax.fori_loop(1, NGRP_D // 2, step, 0)
        sdrain(0)
        sdrain(1)
        plsc.subcore_barrier()

        for off, ln in ZCHUNKS:
            pltpu.sync_copy(acc.at[pl.ds(base + off, ln)],
                            zbuf.at[pl.ds(0, ln)])
            pltpu.sync_copy(zbuf.at[pl.ds(0, ln)],
                            out_ref.at[pl.ds(base + off, ln)])

    @pl.when(c == 0)
    def _():
        run(dst_l, out_l)

    @pl.when(c == 1)
    def _():
        run(dst_f, out_f)


_deg_call = pl.kernel(
    _deg_body,
    out_type=(jax.ShapeDtypeStruct((NP, LANES), jnp.float32),
              jax.ShapeDtypeStruct((NP, LANES), jnp.float32)),
    mesh=_mesh,
    compiler_params=_CP,
    scratch_types=[
        pltpu.VMEM((GEDG_D,), jnp.int32),
        pltpu.VMEM((GEDG_D,), jnp.int32),
        pltpu.VMEM((GEDG_D, LANES), jnp.float32),
        pltpu.VMEM((GEDG, LANES), jnp.float32),
        pltpu.VMEM_SHARED((NP, LANES), jnp.float32),
        pltpu.SemaphoreType.DMA,
        pltpu.SemaphoreType.DMA,
    ],
)


def _prep_edges(src, dst):
    s = src.astype(jnp.int32).reshape(NS, EPT)
    d = dst.astype(jnp.int32).reshape(NS, EPT)
    s = jnp.pad(s, ((0, 0), (0, EPT_PAD - EPT)), constant_values=SRC_PAD)
    d = jnp.pad(d, ((0, 0), (0, EPT_PAD - EPT)), constant_values=DST_PAD)
    packed = jnp.stack([s.reshape(NS, NGRP, GEDG),
                        d.reshape(NS, NGRP, GEDG)], axis=2)
    packed = packed.reshape(NS * NGRP, 2, GEDG)
    packed = jnp.pad(packed, ((0, 8), (0, 0), (0, 0)))
    return packed, d.reshape(NS * EPT_PAD)


def _rowspec(w):
    return pl.BlockSpec((BLK, w), lambda i: (i, 0))


def _fullspec(shape):
    return pl.BlockSpec(shape, lambda i: (0,) * len(shape))


def _ln_relu(y, g, b):
    m = jnp.mean(y, axis=-1, keepdims=True)
    v = jnp.mean((y - m) ** 2, axis=-1, keepdims=True)
    return jnp.maximum((y - m) / jnp.sqrt(v + 1e-5) * g + b, 0.0)


def _conv_tc_body(has_ctx, h0, h1, a0, a1, deg, batch, gctx, W, b, g, bb,
                  o0, o1):
    h = jnp.concatenate([h0[...], h1[...]], axis=1)
    a = jnp.concatenate([a0[...], a1[...]], axis=1)
    d = jnp.maximum(deg[...][:, 0:1], 1.0)
    y = jnp.dot(h + a / d, W[...], preferred_element_type=jnp.float32) + b[...]
    if has_ctx:
        io = lax.broadcasted_iota(jnp.int32, (BLK, G), 1)
        oh = (batch[...] == io).astype(jnp.float32)
        y = y + jnp.dot(oh, gctx[...], preferred_element_type=jnp.float32)
    r = _ln_relu(y, g[...], bb[...])
    o0[...] = r[:, :HH]
    o1[...] = r[:, HH:]


def _make_conv_tc(has_ctx):
    in_specs = [_rowspec(HH)] * 4 + [
        _rowspec(LANES), _rowspec(1), _fullspec((G, H)),
        _fullspec((H, H)), _fullspec((1, H)), _fullspec((1, H)),
        _fullspec((1, H))]
    return pl.pallas_call(
        functools.partial(_conv_tc_body, has_ctx),
        grid=(GRID,),
        in_specs=in_specs,
        out_specs=(_rowspec(HH), _rowspec(HH)),
        out_shape=(jax.ShapeDtypeStruct((NP, HH), jnp.float32),
                   jax.ShapeDtypeStruct((NP, HH), jnp.float32)),
    )


_conv_tc_ctx = _make_conv_tc(True)
_conv_tc_noctx = _make_conv_tc(False)


def _frag_in_body(fx, W, b, o0, o1):
    y = jnp.dot(fx[...], W[...], preferred_element_type=jnp.float32) + b[...]
    o0[...] = y[:, :HH]
    o1[...] = y[:, HH:]


_frag_in = pl.pallas_call(
    _frag_in_body,
    grid=(GRID,),
    in_specs=[_rowspec(4), _fullspec((4, H)), _fullspec((1, H))],
    out_specs=(_rowspec(HH), _rowspec(HH)),
    out_shape=(jax.ShapeDtypeStruct((NP, HH), jnp.float32),
               jax.ShapeDtypeStruct((NP, HH), jnp.float32)),
)


def _linker_in_body(x, nt, batch, gctx, W, b, o0, o1):
    xv = x[...]
    ntv = nt[...]
    y = jnp.dot(xv, W[...][:4, :], preferred_element_type=jnp.float32) + b[...]
    ntc = jnp.clip(ntv, 0, 2)
    for k in range(3):
        y = y + (ntc == k).astype(jnp.float32) * W[...][4 + k:5 + k, :]
    y = y + (ntv > 0).astype(jnp.float32) * W[...][7:8, :]
    io = lax.broadcasted_iota(jnp.int32, (BLK, G), 1)
    oh = (batch[...] == io).astype(jnp.float32)
    y = y + jnp.dot(oh, gctx[...], preferred_element_type=jnp.float32)
    o0[...] = y[:, :HH]
    o1[...] = y[:, HH:]


_linker_in = pl.pallas_call(
    _linker_in_body,
    grid=(GRID,),
    in_specs=[_rowspec(4), _rowspec(1), _rowspec(1), _fullspec((G, H)),
              _fullspec((8, H)), _fullspec((1, H))],
    out_specs=(_rowspec(HH), _rowspec(HH)),
    out_shape=(jax.ShapeDtypeStruct((NP, HH), jnp.float32),
               jax.ShapeDtypeStruct((NP, HH), jnp.float32)),
)


def _frag_tail_body(h0, h1, a0, a1, deg, batch, W, b, g, bb,
                    fW, fb, t, tW1, tb1, tW2, tb2, cW1, cb1, cW2, cb2,
                    gctx_out, sums, cnt):
    i = pl.program_id(0)

    @pl.when(i == 0)
    def _():
        sums[...] = jnp.zeros_like(sums)
        cnt[...] = jnp.zeros_like(cnt)

    h = jnp.concatenate([h0[...], h1[...]], axis=1)
    a = jnp.concatenate([a0[...], a1[...]], axis=1)
    d = jnp.maximum(deg[...][:, 0:1], 1.0)
    y = jnp.dot(h + a / d, W[...], preferred_element_type=jnp.float32) + b[...]
    r = _ln_relu(y, g[...], bb[...])
    io = lax.broadcasted_iota(jnp.int32, (BLK, 2 * G), 1)
    oh = (batch[...] == io).astype(jnp.float32)
    dn = (((0,), (0,)), ((), ()))
    sums[...] += lax.dot_general(oh, r, dn,
                                 preferred_element_type=jnp.float32)
    cnt[...] += lax.dot_general(oh, jnp.ones((BLK, 8), jnp.float32), dn,
                                preferred_element_type=jnp.float32)

    @pl.when(i == GRID - 1)
    def _():
        pooled = sums[...] / jnp.maximum(cnt[...][:, 0:1], 1.0)
        fctx = jnp.dot(pooled, fW[...],
                       preferred_element_type=jnp.float32) + fb[...]
        left = fctx[:G, :]
        right = fctx[G:, :]
        j = lax.broadcasted_iota(jnp.int32, (1, H), 1).astype(jnp.float32)
        freqs = jnp.exp(-math.log(10000.0) * j / 64.0)
        ang = t[...] * freqs
        te = jnp.concatenate([jnp.sin(ang), jnp.cos(ang)], axis=1)
        z = jnp.dot(te, tW1[...], preferred_element_type=jnp.float32) + tb1[...]
        z = z * lax.logistic(z)
        th = jnp.dot(z, tW2[...], preferred_element_type=jnp.float32) + tb2[...]
        ci = jnp.concatenate([left, right, th], axis=1)
        z2 = jnp.dot(ci, cW1[...], preferred_element_type=jnp.float32) + cb1[...]
        z2 = z2 * lax.logistic(z2)
        gctx_out[...] = jnp.dot(z2, cW2[...],
                                preferred_element_type=jnp.float32) + cb2[...]


_frag_tail_tc = pl.pallas_call(
    _frag_tail_body,
    grid=(GRID,),
    in_specs=[_rowspec(HH), _rowspec(HH), _rowspec(HH), _rowspec(HH),
              _rowspec(LANES), _rowspec(1),
              _fullspec((H, H)), _fullspec((1, H)), _fullspec((1, H)),
              _fullspec((1, H)),
              _fullspec((H, H)), _fullspec((1, H)), _fullspec((G, 1)),
              _fullspec((2 * H, H)), _fullspec((1, H)), _fullspec((H, H)),
              _fullspec((1, H)), _fullspec((3 * H, H)), _fullspec((1, H)),
              _fullspec((H, H)), _fullspec((1, H))],
    out_specs=_fullspec((G, H)),
    out_shape=jax.ShapeDtypeStruct((G, H), jnp.float32),
    scratch_shapes=[pltpu.VMEM((2 * G, H), jnp.float32),
                    pltpu.VMEM((2 * G, 8), jnp.float32)],
)


def _linker_tail_body(h0, h1, a0, a1, deg, batch, gctx, W, b, g, bb, oW, ob,
                      out):
    h = jnp.concatenate([h0[...], h1[...]], axis=1)
    a = jnp.concatenate([a0[...], a1[...]], axis=1)
    d = jnp.maximum(deg[...][:, 0:1], 1.0)
    y = jnp.dot(h + a / d, W[...], preferred_element_type=jnp.float32) + b[...]
    io = lax.broadcasted_iota(jnp.int32, (BLK, G), 1)
    oh = (batch[...] == io).astype(jnp.float32)
    y = y + jnp.dot(oh, gctx[...], preferred_element_type=jnp.float32)
    r = _ln_relu(y, g[...], bb[...])
    out[...] = jnp.dot(r, oW[...], preferred_element_type=jnp.float32) + ob[...]


_linker_tail_tc = pl.pallas_call(
    _linker_tail_body,
    grid=(GRID,),
    in_specs=[_rowspec(HH), _rowspec(HH), _rowspec(HH), _rowspec(HH),
              _rowspec(LANES), _rowspec(1), _fullspec((G, H)),
              _fullspec((H, H)), _fullspec((1, H)), _fullspec((1, H)),
              _fullspec((1, H)), _fullspec((H, 4)), _fullspec((1, 4))],
    out_specs=_rowspec(4),
    out_shape=jax.ShapeDtypeStruct((N, 4), jnp.float32),
)


def _row(v):
    return v.reshape(1, -1).astype(jnp.float32)


def kernel(x, t, linker_batch, linker_graph_ptr, linker_node_type,
           linker_edge_index, left_x, left_edge_index, left_batch, right_x,
           right_edge_index, right_batch, params):
    NF = left_x.shape[0]
    p = params
    fp = p['frag']

    fx = jnp.concatenate([left_x, right_x], axis=0)
    f_src = jnp.concatenate([left_edge_index[0], right_edge_index[0] + NF])
    f_dst = jnp.concatenate([left_edge_index[1], right_edge_index[1] + NF])
    f_batch = jnp.concatenate([left_batch, right_batch + G])
    f_idx, f_dflat = _prep_edges(f_src, f_dst)
    l_idx, l_dflat = _prep_edges(linker_edge_index[0], linker_edge_index[1])

    deg_l, deg_f = _deg_call(l_dflat, f_dflat)

    zero_batch = jnp.zeros((N, 1), jnp.int32)
    zero_gctx = jnp.zeros((G, H), jnp.float32)

    f_batch2 = f_batch.astype(jnp.int32).reshape(N, 1)
    nfc = len(fp['conv_W'])
    fh = _frag_in(fx, fp['in_W'], _row(fp['in_b']))
    for i in range(nfc - 1):
        fa = _agg_call(*fh, f_idx)
        fh = _conv_tc_noctx(*fh, *fa, deg_f, zero_batch, zero_gctx,
                            fp['conv_W'][i], _row(fp['conv_b'][i]),
                            _row(fp['ln_g'][i]), _row(fp['ln_b'][i]))
    fa = _agg_call(*fh, f_idx)
    gctx = _frag_tail_tc(*fh, *fa, deg_f, f_batch2,
                         fp['conv_W'][nfc - 1], _row(fp['conv_b'][nfc - 1]),
                         _row(fp['ln_g'][nfc - 1]), _row(fp['ln_b'][nfc - 1]),
                         fp['out_W'], _row(fp['out_b']),
                         t.reshape(G, 1).astype(jnp.float32),
                         p['time_W1'], _row(p['time_b1']),
                         p['time_W2'], _row(p['time_b2']),
                         p['cond_W1'], _row(p['cond_b1']),
                         p['cond_W2'], _row(p['cond_b2']))

    lb = linker_batch.astype(jnp.int32).reshape(N, 1)
    nt = linker_node_type.astype(jnp.int32).reshape(N, 1)
    nlc = len(p['conv_W'])
    h = _linker_in(x[0], nt, lb, gctx, p['in_W'], _row(p['in_b']))
    for i in range(nlc - 1):
        a = _agg_call(*h, l_idx)
        h = _conv_tc_ctx(*h, *a, deg_l, lb, gctx,
                         p['conv_W'][i], _row(p['conv_b'][i]),
                         _row(p['ln_g'][i]), _row(p['ln_b'][i]))
    a = _agg_call(*h, l_idx)
    out = _linker_tail_tc(*h, *a, deg_l, lb, gctx,
                          p['conv_W'][nlc - 1], _row(p['conv_b'][nlc - 1]),
                          _row(p['ln_g'][nlc - 1]), _row(p['ln_b'][nlc - 1]),
                          p['out_W'], _row(p['out_b']))
    return out[None]
